# baseline (speedup 1.0000x reference)
import functools

import jax
import jax.numpy as jnp
from jax import lax
from jax.experimental import pallas as pl
from jax.experimental.pallas import tpu as pltpu
from jax.experimental.pallas import tpu_sc as plsc

_B, _S, _D = 1, 2048, 768
_HQ, _HKV, _HD = 12, 4, 64
_E, _FF, _TOPK, _CAP = 64, 512, 2, 128
_SFF = 2048
_EPS = 1e-6
_T = _B * _S
_NE = _TOPK * _T
_BS = 256
_NSLOT = _E * _CAP
_NW = 32


def _rope(x, cosf, sinf_signed):
    lane = lax.broadcasted_iota(jnp.int32, x.shape, 1)
    left = jnp.roll(x, -32, axis=1)
    right = jnp.roll(x, 32, axis=1)
    swap = jnp.where((lane % _HD) < (_HD // 2), left, right)
    return x * cosf + swap * sinf_signed


def _preattn_body(x_ref, wq_ref, wk_ref, wv_ref, bq_ref, bk_ref, bv_ref,
                  nw_ref, cq_ref, sq_ref, ck_ref, sk_ref,
                  q_ref, k_ref, v_ref):
    x = x_ref[...]
    var = jnp.mean(x * x, axis=1, keepdims=True)
    h = (x * lax.rsqrt(var + _EPS) * nw_ref[...]).astype(jnp.bfloat16)
    q0 = jnp.dot(h, wq_ref[...].astype(jnp.bfloat16),
                 preferred_element_type=jnp.float32) + bq_ref[...]
    k0 = jnp.dot(h, wk_ref[...].astype(jnp.bfloat16),
                 preferred_element_type=jnp.float32) + bk_ref[...]
    v0 = jnp.dot(h, wv_ref[...].astype(jnp.bfloat16),
                 preferred_element_type=jnp.float32) + bv_ref[...]
    q_ref[...] = _rope(q0, cq_ref[...], sq_ref[...])
    k_ref[...] = _rope(k0, ck_ref[...], sk_ref[...])
    v_ref[...] = v0


def _preattn(x2d, p, cosq, sinq, cosk, sink):
    full = lambda shape: pl.BlockSpec(shape, lambda i: (0,) * len(shape))
    row = lambda w: pl.BlockSpec((_BS, w), lambda i: (i, 0))
    return pl.pallas_call(
        _preattn_body,
        grid=(_S // _BS,),
        in_specs=[
            row(_D),
            full((_D, _HQ * _HD)), full((_D, _HKV * _HD)), full((_D, _HKV * _HD)),
            full((1, _HQ * _HD)), full((1, _HKV * _HD)), full((1, _HKV * _HD)),
            full((1, _D)),
            row(_HQ * _HD), row(_HQ * _HD), row(_HKV * _HD), row(_HKV * _HD),
        ],
        out_specs=[row(_HQ * _HD), row(_HKV * _HD), row(_HKV * _HD)],
        out_shape=[
            jax.ShapeDtypeStruct((_S, _HQ * _HD), jnp.float32),
            jax.ShapeDtypeStruct((_S, _HKV * _HD), jnp.float32),
            jax.ShapeDtypeStruct((_S, _HKV * _HD), jnp.float32),
        ],
    )(x2d, p['Wq'], p['Wk'], p['Wv'],
      p['bq'].reshape(1, -1), p['bk'].reshape(1, -1), p['bv'].reshape(1, -1),
      p['attn_norm_w'].reshape(1, -1), cosq, sinq, cosk, sink)


def _attn_body(q_ref, k_ref, v_ref, o_ref, m_ref, l_ref, acc_ref):
    i = pl.program_id(1)
    j = pl.program_id(2)

    @pl.when(j == 0)
    def _():
        m_ref[...] = jnp.full((_BS, 1), -1e30, jnp.float32)
        l_ref[...] = jnp.zeros((_BS, 1), jnp.float32)
        acc_ref[...] = jnp.zeros((_BS, _HD), jnp.float32)

    @pl.when(j <= i)
    def _():
        q = q_ref[0].astype(jnp.bfloat16)
        kb = k_ref[0].astype(jnp.bfloat16)
        vb = v_ref[0].astype(jnp.bfloat16)
        s = lax.dot_general(q, kb, (((1,), (1,)), ((), ())),
                            preferred_element_type=jnp.float32) * (1.0 / 8.0)
        rowi = i * _BS + lax.broadcasted_iota(jnp.int32, s.shape, 0)
        coli = j * _BS + lax.broadcasted_iota(jnp.int32, s.shape, 1)
        s = jnp.where(coli <= rowi, s, -1e9)
        m = m_ref[...]
        mn = jnp.maximum(m, jnp.max(s, axis=1, keepdims=True))
        p = jnp.exp(s - mn)
        corr = jnp.exp(m - mn)
        m_ref[...] = mn
        l_ref[...] = l_ref[...] * corr + jnp.sum(p, axis=1, keepdims=True)
        acc_ref[...] = acc_ref[...] * corr + lax.dot_general(
            p.astype(jnp.bfloat16), vb, (((1,), (0,)), ((), ())),
            preferred_element_type=jnp.float32)

    @pl.when(j == _S // _BS - 1)
    def _():
        o_ref[0] = acc_ref[...] / l_ref[...]


def _attention(qT, kT, vT):
    rep = _HQ // _HKV
    nb = _S // _BS
    kv_map = lambda h, i, j: (h // rep, jnp.minimum(i, j), 0)
    return pl.pallas_call(
        _attn_body,
        grid=(_HQ, nb, nb),
        in_specs=[
            pl.BlockSpec((1, _BS, _HD), lambda h, i, j: (h, i, 0)),
            pl.BlockSpec((1, _BS, _HD), kv_map),
            pl.BlockSpec((1, _BS, _HD), kv_map),
        ],
        out_specs=pl.BlockSpec((1, _BS, _HD), lambda h, i, j: (h, i, 0)),
        out_shape=jax.ShapeDtypeStruct((_HQ, _S, _HD), jnp.float32),
        scratch_shapes=[pltpu.VMEM((_BS, 1), jnp.float32),
                        pltpu.VMEM((_BS, 1), jnp.float32),
                        pltpu.VMEM((_BS, _HD), jnp.float32)],
    )(qT, kT, vT)


def _postattn_body(ao_ref, wo_ref, x_ref, nw_ref, wr_ref, x2_ref, h2_ref, lg_ref):
    x2 = x_ref[...] + jnp.dot(ao_ref[...].astype(jnp.bfloat16),
                              wo_ref[...].astype(jnp.bfloat16),
                              preferred_element_type=jnp.float32)
    var = jnp.mean(x2 * x2, axis=1, keepdims=True)
    h2 = x2 * lax.rsqrt(var + _EPS) * nw_ref[...]
    x2_ref[...] = x2
    h2_ref[...] = h2
    lg_ref[...] = jnp.dot(h2, wr_ref[...], preferred_element_type=jnp.float32)


def _postattn(ao2, x2d, p):
    full = lambda shape: pl.BlockSpec(shape, lambda i: (0,) * len(shape))
    row = lambda w: pl.BlockSpec((_BS, w), lambda i: (i, 0))
    return pl.pallas_call(
        _postattn_body,
        grid=(_S // _BS,),
        in_specs=[row(_D), full((_HQ * _HD, _D)), row(_D), full((1, _D)),
                  full((_D, _E))],
        out_specs=[row(_D), row(_D), row(_E)],
        out_shape=[
            jax.ShapeDtypeStruct((_S, _D), jnp.float32),
            jax.ShapeDtypeStruct((_S, _D), jnp.float32),
            jax.ShapeDtypeStruct((_S, _E), jnp.float32),
        ],
    )(ao2, p['Wo'], x2d, p['ffn_norm_w'].reshape(1, -1), p['Wr'])


def _route_body(lg_ref, dest_ref, wk_ref, aux_ref, counts_ref, psum_ref):
    b = pl.program_id(0)

    @pl.when(b == 0)
    def _():
        counts_ref[...] = jnp.zeros((1, _E), jnp.float32)
        psum_ref[...] = jnp.zeros((1, _E), jnp.float32)

    lg = lg_ref[...]
    m = jnp.max(lg, axis=1, keepdims=True)
    ex = jnp.exp(lg - m)
    prob = ex / jnp.sum(ex, axis=1, keepdims=True)

    @pl.when(b < _S // _BS)
    def _():
        psum_ref[...] += jnp.sum(prob, axis=0, keepdims=True)

    ie = lax.broadcasted_iota(jnp.int32, (_BS, _E), 1)
    m1 = jnp.max(prob, axis=1, keepdims=True)
    i1 = jnp.min(jnp.where(prob >= m1, ie, _E), axis=1, keepdims=True)
    p2 = jnp.where(ie == i1, -1.0, prob)
    m2 = jnp.max(p2, axis=1, keepdims=True)
    i2 = jnp.min(jnp.where(p2 >= m2, ie, _E), axis=1, keepdims=True)
    den = m1 + m2
    c = b // (_S // _BS)
    fe = jnp.where(c == 0, i1, i2)
    w = jnp.where(c == 0, m1, m2) / den
    oh = (ie == fe).astype(jnp.float32)
    ri = lax.broadcasted_iota(jnp.int32, (_BS, _BS), 0)
    ci = lax.broadcasted_iota(jnp.int32, (_BS, _BS), 1)
    ltri = (ci < ri).astype(jnp.float32)
    before = jnp.dot(ltri, oh, preferred_element_type=jnp.float32)
    pos = jnp.sum((counts_ref[...] + before) * oh, axis=1,
                  keepdims=True).astype(jnp.int32)
    counts_ref[...] += jnp.sum(oh, axis=0, keepdims=True)
    keep = pos < _CAP
    posc = jnp.minimum(pos, _CAP - 1)
    entry = b * _BS + lax.broadcasted_iota(jnp.int32, (_BS, 1), 0)
    dest_ref[...] = jnp.where(keep, fe * _CAP + posc, _NSLOT + (entry & 7))
    wk_ref[...] = jnp.where(keep, w, 0.0)

    @pl.when(b == _NE // _BS - 1)
    def _():
        aux_ref[...] = (_E * jnp.sum(counts_ref[...] * psum_ref[...],
                                     axis=1, keepdims=True)
                        / (float(_NE) * float(_T)))


def _route(logits):
    nb = _NE // _BS
    return pl.pallas_call(
        _route_body,
        grid=(nb,),
        in_specs=[pl.BlockSpec((_BS, _E), lambda b: (b % (_S // _BS), 0))],
        out_specs=[
            pl.BlockSpec((_BS, 1), lambda b: (b, 0)),
            pl.BlockSpec((_BS, 1), lambda b: (b, 0)),
            pl.BlockSpec((1, 1), lambda b: (0, 0)),
        ],
        out_shape=[
            jax.ShapeDtypeStruct((_NE, 1), jnp.int32),
            jax.ShapeDtypeStruct((_NE, 1), jnp.float32),
            jax.ShapeDtypeStruct((1, 1), jnp.float32),
        ],
        scratch_shapes=[pltpu.VMEM((1, _E), jnp.float32),
                        pltpu.VMEM((1, _E), jnp.float32)],
    )(logits)


def _sc_mesh():
    return plsc.VectorSubcoreMesh(core_axis_name="c", subcore_axis_name="s")


def _build_tables(dest, wk):
    per_w = _NSLOT // _NW

    @functools.partial(
        pl.kernel,
        out_type=(jax.ShapeDtypeStruct((_NSLOT,), jnp.int32),
                  jax.ShapeDtypeStruct((_NSLOT,), jnp.float32)),
        mesh=_sc_mesh(),
        scratch_types=[pltpu.VMEM((_NE,), jnp.int32),
                       pltpu.VMEM((_NE,), jnp.float32),
                       pltpu.VMEM((per_w,), jnp.int32),
                       pltpu.VMEM((per_w,), jnp.float32)],
        compiler_params=pltpu.CompilerParams(needs_layout_passes=False),
    )
    def k(dest_hbm, wk_hbm, src_hbm, sw_hbm, dest_v, wk_v, src_l, sw_l):
        wid = lax.axis_index("s") * 2 + lax.axis_index("c")
        lo = wid * per_w
        pltpu.sync_copy(dest_hbm, dest_v)
        pltpu.sync_copy(wk_hbm, wk_v)
        for i in range(per_w // 16):
            evec = lo + i * 16 + jnp.arange(16, dtype=jnp.int32)
            src_l[pl.ds(i * 16, 16)] = evec & (_T - 1)
            sw_l[pl.ds(i * 16, 16)] = jnp.zeros((16,), jnp.float32)

        def body(i, carry):
            d = dest_v[pl.ds(i * 16, 16)]
            w = wk_v[pl.ds(i * 16, 16)]
            evec = i * 16 + jnp.arange(16, dtype=jnp.int32)
            tok = jnp.where(evec >= _T, evec - _T, evec)
            msk = (d >= lo) & (d < lo + per_w)
            plsc.store_scatter(src_l, [d - lo], tok, mask=msk)
            plsc.store_scatter(sw_l, [d - lo], w, mask=msk)
            return carry

        lax.fori_loop(0, _NE // 16, body, 0)
        pltpu.sync_copy(src_l, src_hbm.at[pl.ds(lo, per_w)])
        pltpu.sync_copy(sw_l, sw_hbm.at[pl.ds(lo, per_w)])

    return k(dest, wk)


def _sc_gather(table, idx, n_rows, chunk):
    per_w = n_rows // _NW
    nch = per_w // chunk

    @functools.partial(
        pl.kernel,
        out_type=jax.ShapeDtypeStruct((n_rows, _D), jnp.float32),
        mesh=_sc_mesh(),
        scratch_types=[pltpu.VMEM((chunk,), jnp.int32),
                       pltpu.VMEM((chunk, _D), jnp.float32),
                       pltpu.SemaphoreType.DMA],
        compiler_params=pltpu.CompilerParams(needs_layout_passes=False),
    )
    def k(table_hbm, idx_hbm, out_hbm, idx_v, rows_v, sem):
        wid = lax.axis_index("s") * 2 + lax.axis_index("c")
        base = wid * per_w
        for ci in range(nch):
            off = base + ci * chunk
            pltpu.sync_copy(idx_hbm.at[pl.ds(off, chunk)], idx_v)
            pltpu.async_copy(table_hbm.at[idx_v], rows_v, sem).wait()
            pltpu.sync_copy(rows_v, out_hbm.at[pl.ds(off, chunk)])

    return k(table, idx)


def _expert_body(ein_ref, wg_ref, wu_ref, wd_ref, sw_ref, out_ref):
    xin = ein_ref[0].astype(jnp.bfloat16)
    g = jnp.dot(xin, wg_ref[0].astype(jnp.bfloat16),
                preferred_element_type=jnp.float32)
    u = jnp.dot(xin, wu_ref[0].astype(jnp.bfloat16),
                preferred_element_type=jnp.float32)
    a = (g * lax.logistic(g) * u).astype(jnp.bfloat16)
    o = jnp.dot(a, wd_ref[0].astype(jnp.bfloat16),
                preferred_element_type=jnp.float32)
    out_ref[0] = o * sw_ref[0]


def _experts(ein3, sw3, p):
    return pl.pallas_call(
        _expert_body,
        grid=(_E,),
        in_specs=[
            pl.BlockSpec((1, _CAP, _D), lambda e: (e, 0, 0)),
            pl.BlockSpec((1, _D, _FF), lambda e: (e, 0, 0)),
            pl.BlockSpec((1, _D, _FF), lambda e: (e, 0, 0)),
            pl.BlockSpec((1, _FF, _D), lambda e: (e, 0, 0)),
            pl.BlockSpec((1, _CAP, 1), lambda e: (e, 0, 0)),
        ],
        out_specs=pl.BlockSpec((1, _CAP, _D), lambda e: (e, 0, 0)),
        out_shape=jax.ShapeDtypeStruct((_E, _CAP, _D), jnp.float32),
    )(ein3, p['Wg'], p['Wu'], p['Wd'], sw3)


def _final_body(x2_ref, ta_ref, tb_ref, h2_ref, sg_ref, su_ref, sd_ref, o_ref):
    h2 = h2_ref[...].astype(jnp.bfloat16)
    g = jnp.dot(h2, sg_ref[...].astype(jnp.bfloat16),
                preferred_element_type=jnp.float32)
    u = jnp.dot(h2, su_ref[...].astype(jnp.bfloat16),
                preferred_element_type=jnp.float32)
    sh = jnp.dot((g * lax.logistic(g) * u).astype(jnp.bfloat16),
                 sd_ref[...].astype(jnp.bfloat16),
                 preferred_element_type=jnp.float32)
    o_ref[...] = x2_ref[...] + ta_ref[...] + tb_ref[...] + sh


def _final(x2, tok, h2, p):
    full = lambda shape: pl.BlockSpec(shape, lambda i: (0,) * len(shape))
    row = lambda w: pl.BlockSpec((_BS, w), lambda i: (i, 0))
    nb = _S // _BS
    return pl.pallas_call(
        _final_body,
        grid=(nb,),
        in_specs=[
            row(_D),
            pl.BlockSpec((_BS, _D), lambda i: (i, 0)),
            pl.BlockSpec((_BS, _D), lambda i: (i + nb, 0)),
            row(_D),
            full((_D, _SFF)), full((_D, _SFF)), full((_SFF, _D)),
        ],
        out_specs=row(_D),
        out_shape=jax.ShapeDtypeStruct((_S, _D), jnp.float32),
    )(x2, tok, tok, h2, p['Sg'], p['Su'], p['Sd'])


def kernel(x, rope_cos, rope_sin, params):
    p = params
    x2d = x.reshape(_S, _D)
    sign = jnp.concatenate([-jnp.ones((_HD // 2,), jnp.float32),
                            jnp.ones((_HD // 2,), jnp.float32)])
    cosq = jnp.tile(rope_cos, (1, _HQ))
    sinq = jnp.tile(rope_sin * sign[None, :], (1, _HQ))
    cosk = jnp.tile(rope_cos, (1, _HKV))
    sink = jnp.tile(rope_sin * sign[None, :], (1, _HKV))

    q, kk, v = _preattn(x2d, p, cosq, sinq, cosk, sink)
    kv_k = kk.reshape(_B, _S, _HKV, _HD)
    kv_v = v.reshape(_B, _S, _HKV, _HD)

    qT = q.reshape(_S, _HQ, _HD).transpose(1, 0, 2)
    kT = kk.reshape(_S, _HKV, _HD).transpose(1, 0, 2)
    vT = v.reshape(_S, _HKV, _HD).transpose(1, 0, 2)
    ao = _attention(qT, kT, vT)
    ao2 = ao.transpose(1, 0, 2).reshape(_S, _HQ * _HD)

    x2, h2, logits = _postattn(ao2, x2d, p)
    dest2, wk2, aux = _route(logits)
    dest = dest2.reshape(_NE)
    wk = wk2.reshape(_NE)

    src, sw = _build_tables(dest, wk)

    ein = _sc_gather(h2, src, _NSLOT, 64)
    eout = _experts(ein.reshape(_E, _CAP, _D), sw.reshape(_E, _CAP, 1), p)
    eoutp = jnp.concatenate([eout.reshape(_NSLOT, _D),
                             jnp.zeros((8, _D), jnp.float32)], axis=0)
    tok = _sc_gather(eoutp, dest, _NE, 64)

    out = _final(x2, tok, h2, p)
    return out.reshape(_B, _S, _D), (kv_k, kv_v), aux.reshape(())


# speedup vs baseline: 1.6574x; 1.6574x over previous
import functools

import jax
import jax.numpy as jnp
from jax import lax
from jax.experimental import pallas as pl
from jax.experimental.pallas import tpu as pltpu
from jax.experimental.pallas import tpu_sc as plsc

_B, _S, _D = 1, 2048, 768
_HQ, _HKV, _HD = 12, 4, 64
_E, _FF, _TOPK, _CAP = 64, 512, 2, 128
_SFF = 2048
_EPS = 1e-6
_T = _B * _S
_NE = _TOPK * _T
_BS = 256
_NSLOT = _E * _CAP
_NW = 32


def _rope(x, cosf, sinf_signed):
    lane = lax.broadcasted_iota(jnp.int32, x.shape, 1)
    left = jnp.roll(x, -32, axis=1)
    right = jnp.roll(x, 32, axis=1)
    swap = jnp.where((lane % _HD) < (_HD // 2), left, right)
    return x * cosf + swap * sinf_signed


def _preattn_body(x_ref, wq_ref, wk_ref, wv_ref, bq_ref, bk_ref, bv_ref,
                  nw_ref, cq_ref, sq_ref, ck_ref, sk_ref,
                  q_ref, k_ref, v_ref):
    x = x_ref[...]
    var = jnp.mean(x * x, axis=1, keepdims=True)
    h = (x * lax.rsqrt(var + _EPS) * nw_ref[...]).astype(jnp.bfloat16)
    q0 = jnp.dot(h, wq_ref[...].astype(jnp.bfloat16),
                 preferred_element_type=jnp.float32) + bq_ref[...]
    k0 = jnp.dot(h, wk_ref[...].astype(jnp.bfloat16),
                 preferred_element_type=jnp.float32) + bk_ref[...]
    v0 = jnp.dot(h, wv_ref[...].astype(jnp.bfloat16),
                 preferred_element_type=jnp.float32) + bv_ref[...]
    q_ref[...] = _rope(q0, cq_ref[...], sq_ref[...])
    k_ref[...] = _rope(k0, ck_ref[...], sk_ref[...])
    v_ref[...] = v0


def _preattn(x2d, p, cosq, sinq, cosk, sink):
    full = lambda shape: pl.BlockSpec(shape, lambda i: (0,) * len(shape))
    row = lambda w: pl.BlockSpec((_BS, w), lambda i: (i, 0))
    return pl.pallas_call(
        _preattn_body,
        grid=(_S // _BS,),
        in_specs=[
            row(_D),
            full((_D, _HQ * _HD)), full((_D, _HKV * _HD)), full((_D, _HKV * _HD)),
            full((1, _HQ * _HD)), full((1, _HKV * _HD)), full((1, _HKV * _HD)),
            full((1, _D)),
            row(_HQ * _HD), row(_HQ * _HD), row(_HKV * _HD), row(_HKV * _HD),
        ],
        out_specs=[row(_HQ * _HD), row(_HKV * _HD), row(_HKV * _HD)],
        out_shape=[
            jax.ShapeDtypeStruct((_S, _HQ * _HD), jnp.float32),
            jax.ShapeDtypeStruct((_S, _HKV * _HD), jnp.float32),
            jax.ShapeDtypeStruct((_S, _HKV * _HD), jnp.float32),
        ],
    )(x2d, p['Wq'], p['Wk'], p['Wv'],
      p['bq'].reshape(1, -1), p['bk'].reshape(1, -1), p['bv'].reshape(1, -1),
      p['attn_norm_w'].reshape(1, -1), cosq, sinq, cosk, sink)


def _attn_body(q_ref, k_ref, v_ref, o_ref):
    i = pl.program_id(1)
    q = q_ref[0].astype(jnp.bfloat16)
    k = k_ref[0].astype(jnp.bfloat16)
    v = v_ref[0].astype(jnp.bfloat16)
    s = lax.dot_general(q, k, (((1,), (1,)), ((), ())),
                        preferred_element_type=jnp.float32) * (1.0 / 8.0)
    rowi = i * _BS + lax.broadcasted_iota(jnp.int32, s.shape, 0)
    coli = lax.broadcasted_iota(jnp.int32, s.shape, 1)
    s = jnp.where(coli <= rowi, s, -1e9)
    m = jnp.max(s, axis=1, keepdims=True)
    e = jnp.exp(s - m)
    a = (e / jnp.sum(e, axis=1, keepdims=True)).astype(jnp.bfloat16)
    o_ref[0] = lax.dot_general(a, v, (((1,), (0,)), ((), ())),
                               preferred_element_type=jnp.float32)


def _attention(qT, kT, vT):
    rep = _HQ // _HKV
    return pl.pallas_call(
        _attn_body,
        grid=(_HQ, _S // _BS),
        in_specs=[
            pl.BlockSpec((1, _BS, _HD), lambda h, i: (h, i, 0)),
            pl.BlockSpec((1, _S, _HD), lambda h, i: (h // rep, 0, 0)),
            pl.BlockSpec((1, _S, _HD), lambda h, i: (h // rep, 0, 0)),
        ],
        out_specs=pl.BlockSpec((1, _BS, _HD), lambda h, i: (h, i, 0)),
        out_shape=jax.ShapeDtypeStruct((_HQ, _S, _HD), jnp.float32),
    )(qT, kT, vT)


def _postattn_body(ao_ref, wo_ref, x_ref, nw_ref, wr_ref, x2_ref, h2_ref, lg_ref):
    x2 = x_ref[...] + jnp.dot(ao_ref[...].astype(jnp.bfloat16),
                              wo_ref[...].astype(jnp.bfloat16),
                              preferred_element_type=jnp.float32)
    var = jnp.mean(x2 * x2, axis=1, keepdims=True)
    h2 = x2 * lax.rsqrt(var + _EPS) * nw_ref[...]
    x2_ref[...] = x2
    h2_ref[...] = h2
    lg_ref[...] = jnp.dot(h2, wr_ref[...], preferred_element_type=jnp.float32)


def _postattn(ao2, x2d, p):
    full = lambda shape: pl.BlockSpec(shape, lambda i: (0,) * len(shape))
    row = lambda w: pl.BlockSpec((_BS, w), lambda i: (i, 0))
    return pl.pallas_call(
        _postattn_body,
        grid=(_S // _BS,),
        in_specs=[row(_D), full((_HQ * _HD, _D)), row(_D), full((1, _D)),
                  full((_D, _E))],
        out_specs=[row(_D), row(_D), row(_E)],
        out_shape=[
            jax.ShapeDtypeStruct((_S, _D), jnp.float32),
            jax.ShapeDtypeStruct((_S, _D), jnp.float32),
            jax.ShapeDtypeStruct((_S, _E), jnp.float32),
        ],
    )(ao2, p['Wo'], x2d, p['ffn_norm_w'].reshape(1, -1), p['Wr'])


def _route_body(lg_ref, dest_ref, wk_ref, aux_ref, counts_ref, psum_ref):
    b = pl.program_id(0)

    @pl.when(b == 0)
    def _():
        counts_ref[...] = jnp.zeros((1, _E), jnp.float32)
        psum_ref[...] = jnp.zeros((1, _E), jnp.float32)

    lg = lg_ref[...]
    m = jnp.max(lg, axis=1, keepdims=True)
    ex = jnp.exp(lg - m)
    prob = ex / jnp.sum(ex, axis=1, keepdims=True)

    @pl.when(b < _S // _BS)
    def _():
        psum_ref[...] += jnp.sum(prob, axis=0, keepdims=True)

    ie = lax.broadcasted_iota(jnp.int32, (_BS, _E), 1)
    m1 = jnp.max(prob, axis=1, keepdims=True)
    i1 = jnp.min(jnp.where(prob >= m1, ie, _E), axis=1, keepdims=True)
    p2 = jnp.where(ie == i1, -1.0, prob)
    m2 = jnp.max(p2, axis=1, keepdims=True)
    i2 = jnp.min(jnp.where(p2 >= m2, ie, _E), axis=1, keepdims=True)
    den = m1 + m2
    c = b // (_S // _BS)
    fe = jnp.where(c == 0, i1, i2)
    w = jnp.where(c == 0, m1, m2) / den
    oh = (ie == fe).astype(jnp.float32)
    ri = lax.broadcasted_iota(jnp.int32, (_BS, _BS), 0)
    ci = lax.broadcasted_iota(jnp.int32, (_BS, _BS), 1)
    ltri = (ci < ri).astype(jnp.float32)
    before = jnp.dot(ltri, oh, preferred_element_type=jnp.float32)
    pos = jnp.sum((counts_ref[...] + before) * oh, axis=1,
                  keepdims=True).astype(jnp.int32)
    counts_ref[...] += jnp.sum(oh, axis=0, keepdims=True)
    keep = pos < _CAP
    posc = jnp.minimum(pos, _CAP - 1)
    entry = b * _BS + lax.broadcasted_iota(jnp.int32, (_BS, 1), 0)
    dest_ref[...] = jnp.where(keep, fe * _CAP + posc, _NSLOT + (entry & (_CAP - 1)))
    wk_ref[...] = jnp.where(keep, w, 0.0)

    @pl.when(b == _NE // _BS - 1)
    def _():
        aux_ref[...] = (_E * jnp.sum(counts_ref[...] * psum_ref[...],
                                     axis=1, keepdims=True)
                        / (float(_NE) * float(_T)))


def _route(logits):
    nb = _NE // _BS
    return pl.pallas_call(
        _route_body,
        grid=(nb,),
        in_specs=[pl.BlockSpec((_BS, _E), lambda b: (b % (_S // _BS), 0))],
        out_specs=[
            pl.BlockSpec((_BS, 1), lambda b: (b, 0)),
            pl.BlockSpec((_BS, 1), lambda b: (b, 0)),
            pl.BlockSpec((1, 1), lambda b: (0, 0)),
        ],
        out_shape=[
            jax.ShapeDtypeStruct((_NE, 1), jnp.int32),
            jax.ShapeDtypeStruct((_NE, 1), jnp.float32),
            jax.ShapeDtypeStruct((1, 1), jnp.float32),
        ],
        scratch_shapes=[pltpu.VMEM((1, _E), jnp.float32),
                        pltpu.VMEM((1, _E), jnp.float32)],
    )(logits)


def _sc_mesh():
    return plsc.VectorSubcoreMesh(core_axis_name="c", subcore_axis_name="s")


def _build_tables(dest, wk):
    per_w = _NSLOT // _NW

    @functools.partial(
        pl.kernel,
        out_type=(jax.ShapeDtypeStruct((_NSLOT,), jnp.int32),
                  jax.ShapeDtypeStruct((_NSLOT,), jnp.float32)),
        mesh=_sc_mesh(),
        scratch_types=[pltpu.VMEM((_NE,), jnp.int32),
                       pltpu.VMEM((_NE,), jnp.float32),
                       pltpu.VMEM((per_w,), jnp.int32),
                       pltpu.VMEM((per_w,), jnp.float32)],
        compiler_params=pltpu.CompilerParams(needs_layout_passes=False),
    )
    def k(dest_hbm, wk_hbm, src_hbm, sw_hbm, dest_v, wk_v, src_l, sw_l):
        wid = lax.axis_index("s") * 2 + lax.axis_index("c")
        lo = wid * per_w
        pltpu.sync_copy(dest_hbm, dest_v)
        pltpu.sync_copy(wk_hbm, wk_v)
        for i in range(per_w // 16):
            evec = lo + i * 16 + jnp.arange(16, dtype=jnp.int32)
            src_l[pl.ds(i * 16, 16)] = evec & (_T - 1)
            sw_l[pl.ds(i * 16, 16)] = jnp.zeros((16,), jnp.float32)

        def body(i, carry):
            d = dest_v[pl.ds(i * 16, 16)]
            w = wk_v[pl.ds(i * 16, 16)]
            evec = i * 16 + jnp.arange(16, dtype=jnp.int32)
            tok = jnp.where(evec >= _T, evec - _T, evec)
            msk = (d >= lo) & (d < lo + per_w)
            plsc.store_scatter(src_l, [d - lo], tok, mask=msk)
            plsc.store_scatter(sw_l, [d - lo], w, mask=msk)
            return carry

        lax.fori_loop(0, _NE // 16, body, 0)
        pltpu.sync_copy(src_l, src_hbm.at[pl.ds(lo, per_w)])
        pltpu.sync_copy(sw_l, sw_hbm.at[pl.ds(lo, per_w)])

    return k(dest, wk)


def _sc_gather(table, idx, n_rows, chunk):
    per_w = n_rows // _NW
    nch = per_w // chunk

    @functools.partial(
        pl.kernel,
        out_type=jax.ShapeDtypeStruct((n_rows, _D), jnp.float32),
        mesh=_sc_mesh(),
        scratch_types=[pltpu.VMEM((chunk,), jnp.int32),
                       pltpu.VMEM((chunk, _D), jnp.float32),
                       pltpu.SemaphoreType.DMA],
        compiler_params=pltpu.CompilerParams(needs_layout_passes=False),
    )
    def k(table_hbm, idx_hbm, out_hbm, idx_v, rows_v, sem):
        wid = lax.axis_index("s") * 2 + lax.axis_index("c")
        base = wid * per_w
        for ci in range(nch):
            off = base + ci * chunk
            pltpu.sync_copy(idx_hbm.at[pl.ds(off, chunk)], idx_v)
            pltpu.async_copy(table_hbm.at[idx_v], rows_v, sem).wait()
            pltpu.sync_copy(rows_v, out_hbm.at[pl.ds(off, chunk)])

    return k(table, idx)


def _expert_body(ein_ref, wg_ref, wu_ref, wd_ref, sw_ref, out_ref):
    e = pl.program_id(0)

    @pl.when(e < _E)
    def _():
        xin = ein_ref[0].astype(jnp.bfloat16)
        g = jnp.dot(xin, wg_ref[0].astype(jnp.bfloat16),
                    preferred_element_type=jnp.float32)
        u = jnp.dot(xin, wu_ref[0].astype(jnp.bfloat16),
                    preferred_element_type=jnp.float32)
        a = (g * lax.logistic(g) * u).astype(jnp.bfloat16)
        o = jnp.dot(a, wd_ref[0].astype(jnp.bfloat16),
                    preferred_element_type=jnp.float32)
        out_ref[0] = o * sw_ref[0]

    @pl.when(e == _E)
    def _():
        out_ref[0] = jnp.zeros((_CAP, _D), jnp.float32)


def _experts(ein3, sw3, p):
    cl = lambda e: (jnp.minimum(e, _E - 1), 0, 0)
    return pl.pallas_call(
        _expert_body,
        grid=(_E + 1,),
        in_specs=[
            pl.BlockSpec((1, _CAP, _D), cl),
            pl.BlockSpec((1, _D, _FF), cl),
            pl.BlockSpec((1, _D, _FF), cl),
            pl.BlockSpec((1, _FF, _D), cl),
            pl.BlockSpec((1, _CAP, 1), cl),
        ],
        out_specs=pl.BlockSpec((1, _CAP, _D), lambda e: (e, 0, 0)),
        out_shape=jax.ShapeDtypeStruct((_E + 1, _CAP, _D), jnp.float32),
    )(ein3, p['Wg'], p['Wu'], p['Wd'], sw3)


def _shared_body(h2_ref, sg_ref, su_ref, sd_ref, o_ref):
    h2 = h2_ref[...].astype(jnp.bfloat16)
    g = jnp.dot(h2, sg_ref[...].astype(jnp.bfloat16),
                preferred_element_type=jnp.float32)
    u = jnp.dot(h2, su_ref[...].astype(jnp.bfloat16),
                preferred_element_type=jnp.float32)
    o_ref[...] = jnp.dot((g * lax.logistic(g) * u).astype(jnp.bfloat16),
                         sd_ref[...].astype(jnp.bfloat16),
                         preferred_element_type=jnp.float32)


def _shared(h2, p):
    full = lambda shape: pl.BlockSpec(shape, lambda i: (0,) * len(shape))
    row = lambda w: pl.BlockSpec((_BS, w), lambda i: (i, 0))
    return pl.pallas_call(
        _shared_body,
        grid=(_S // _BS,),
        in_specs=[row(_D), full((_D, _SFF)), full((_D, _SFF)),
                  full((_SFF, _D))],
        out_specs=row(_D),
        out_shape=jax.ShapeDtypeStruct((_S, _D), jnp.float32),
    )(h2, p['Sg'], p['Su'], p['Sd'])


def _final_body(x2_ref, ta_ref, tb_ref, sh_ref, o_ref):
    o_ref[...] = x2_ref[...] + ta_ref[...] + tb_ref[...] + sh_ref[...]


def _final(x2, tok, shared):
    row = lambda w: pl.BlockSpec((_BS, w), lambda i: (i, 0))
    nb = _S // _BS
    return pl.pallas_call(
        _final_body,
        grid=(nb,),
        in_specs=[
            row(_D),
            pl.BlockSpec((_BS, _D), lambda i: (i, 0)),
            pl.BlockSpec((_BS, _D), lambda i: (i + nb, 0)),
            row(_D),
        ],
        out_specs=row(_D),
        out_shape=jax.ShapeDtypeStruct((_S, _D), jnp.float32),
    )(x2, tok, tok, shared)


def kernel(x, rope_cos, rope_sin, params):
    p = params
    x2d = x.reshape(_S, _D)
    sign = jnp.concatenate([-jnp.ones((_HD // 2,), jnp.float32),
                            jnp.ones((_HD // 2,), jnp.float32)])
    cosq = jnp.tile(rope_cos, (1, _HQ))
    sinq = jnp.tile(rope_sin * sign[None, :], (1, _HQ))
    cosk = jnp.tile(rope_cos, (1, _HKV))
    sink = jnp.tile(rope_sin * sign[None, :], (1, _HKV))

    q, kk, v = _preattn(x2d, p, cosq, sinq, cosk, sink)
    kv_k = kk.reshape(_B, _S, _HKV, _HD)
    kv_v = v.reshape(_B, _S, _HKV, _HD)

    qT = q.reshape(_S, _HQ, _HD).transpose(1, 0, 2)
    kT = kk.reshape(_S, _HKV, _HD).transpose(1, 0, 2)
    vT = v.reshape(_S, _HKV, _HD).transpose(1, 0, 2)
    ao = _attention(qT, kT, vT)
    ao2 = ao.transpose(1, 0, 2).reshape(_S, _HQ * _HD)

    x2, h2, logits = _postattn(ao2, x2d, p)
    shared = _shared(h2, p)
    dest2, wk2, aux = _route(logits)
    dest = dest2.reshape(_NE)
    wk = wk2.reshape(_NE)

    src, sw = _build_tables(dest, wk)

    ein = _sc_gather(h2, src, _NSLOT, 64)
    eout = _experts(ein.reshape(_E, _CAP, _D), sw.reshape(_E, _CAP, 1), p)
    eoutp = eout.reshape((_E + 1) * _CAP, _D)
    tok = _sc_gather(eoutp, dest, _NE, 64)

    out = _final(x2, tok, shared)
    return out.reshape(_B, _S, _D), (kv_k, kv_v), aux.reshape(())


# speedup vs baseline: 1.7637x; 1.0641x over previous
import functools

import jax
import jax.numpy as jnp
from jax import lax
from jax.experimental import pallas as pl
from jax.experimental.pallas import tpu as pltpu
from jax.experimental.pallas import tpu_sc as plsc

_B, _S, _D = 1, 2048, 768
_HQ, _HKV, _HD = 12, 4, 64
_E, _FF, _TOPK, _CAP = 64, 512, 2, 128
_SFF = 2048
_EPS = 1e-6
_T = _B * _S
_NE = _TOPK * _T
_BS = 256
_NSLOT = _E * _CAP
_NW = 32


def _rope(x, cosf, sinf_signed):
    lane = lax.broadcasted_iota(jnp.int32, x.shape, 1)
    left = jnp.roll(x, -32, axis=1)
    right = jnp.roll(x, 32, axis=1)
    swap = jnp.where((lane % _HD) < (_HD // 2), left, right)
    return x * cosf + swap * sinf_signed


def _preattn_body(x_ref, wq_ref, wk_ref, wv_ref, bq_ref, bk_ref, bv_ref,
                  nw_ref, cq_ref, sq_ref, ck_ref, sk_ref,
                  q_ref, k_ref, v_ref):
    x = x_ref[...]
    var = jnp.mean(x * x, axis=1, keepdims=True)
    h = (x * lax.rsqrt(var + _EPS) * nw_ref[...]).astype(jnp.bfloat16)
    q0 = jnp.dot(h, wq_ref[...].astype(jnp.bfloat16),
                 preferred_element_type=jnp.float32) + bq_ref[...]
    k0 = jnp.dot(h, wk_ref[...].astype(jnp.bfloat16),
                 preferred_element_type=jnp.float32) + bk_ref[...]
    v0 = jnp.dot(h, wv_ref[...].astype(jnp.bfloat16),
                 preferred_element_type=jnp.float32) + bv_ref[...]
    q_ref[...] = _rope(q0, cq_ref[...], sq_ref[...])
    k_ref[...] = _rope(k0, ck_ref[...], sk_ref[...])
    v_ref[...] = v0


def _preattn(x2d, p, cosq, sinq, cosk, sink):
    full = lambda shape: pl.BlockSpec(shape, lambda i: (0,) * len(shape))
    row = lambda w: pl.BlockSpec((_BS, w), lambda i: (i, 0))
    return pl.pallas_call(
        _preattn_body,
        grid=(_S // _BS,),
        in_specs=[
            row(_D),
            full((_D, _HQ * _HD)), full((_D, _HKV * _HD)), full((_D, _HKV * _HD)),
            full((1, _HQ * _HD)), full((1, _HKV * _HD)), full((1, _HKV * _HD)),
            full((1, _D)),
            row(_HQ * _HD), row(_HQ * _HD), row(_HKV * _HD), row(_HKV * _HD),
        ],
        out_specs=[row(_HQ * _HD), row(_HKV * _HD), row(_HKV * _HD)],
        out_shape=[
            jax.ShapeDtypeStruct((_S, _HQ * _HD), jnp.float32),
            jax.ShapeDtypeStruct((_S, _HKV * _HD), jnp.float32),
            jax.ShapeDtypeStruct((_S, _HKV * _HD), jnp.float32),
        ],
    )(x2d, p['Wq'], p['Wk'], p['Wv'],
      p['bq'].reshape(1, -1), p['bk'].reshape(1, -1), p['bv'].reshape(1, -1),
      p['attn_norm_w'].reshape(1, -1), cosq, sinq, cosk, sink)


def _attn_body(q_ref, k_ref, v_ref, o_ref):
    i = pl.program_id(1)
    q = q_ref[0].astype(jnp.bfloat16)

    def branch(width):
        kb = k_ref[0, 0:width, :].astype(jnp.bfloat16)
        vb = v_ref[0, 0:width, :].astype(jnp.bfloat16)
        s = lax.dot_general(q, kb, (((1,), (1,)), ((), ())),
                            preferred_element_type=jnp.float32) * (1.0 / 8.0)
        rowi = i * _BS + lax.broadcasted_iota(jnp.int32, s.shape, 0)
        coli = lax.broadcasted_iota(jnp.int32, s.shape, 1)
        s = jnp.where(coli <= rowi, s, -1e9)
        m = jnp.max(s, axis=1, keepdims=True)
        e = jnp.exp(s - m)
        a = (e / jnp.sum(e, axis=1, keepdims=True)).astype(jnp.bfloat16)
        o_ref[0] = lax.dot_general(a, vb, (((1,), (0,)), ((), ())),
                                   preferred_element_type=jnp.float32)

    for bi in range(4):
        width = (bi + 1) * 2 * _BS

        @pl.when((i >= 2 * bi) & (i < 2 * bi + 2))
        def _(width=width):
            branch(width)


def _attention(qT, kT, vT):
    rep = _HQ // _HKV
    return pl.pallas_call(
        _attn_body,
        grid=(_HQ, _S // _BS),
        in_specs=[
            pl.BlockSpec((1, _BS, _HD), lambda h, i: (h, i, 0)),
            pl.BlockSpec((1, _S, _HD), lambda h, i: (h // rep, 0, 0)),
            pl.BlockSpec((1, _S, _HD), lambda h, i: (h // rep, 0, 0)),
        ],
        out_specs=pl.BlockSpec((1, _BS, _HD), lambda h, i: (h, i, 0)),
        out_shape=jax.ShapeDtypeStruct((_HQ, _S, _HD), jnp.float32),
    )(qT, kT, vT)


def _postattn_body(ao_ref, wo_ref, x_ref, nw_ref, wr_ref, x2_ref, h2_ref, lg_ref):
    x2 = x_ref[...] + jnp.dot(ao_ref[...].astype(jnp.bfloat16),
                              wo_ref[...].astype(jnp.bfloat16),
                              preferred_element_type=jnp.float32)
    var = jnp.mean(x2 * x2, axis=1, keepdims=True)
    h2 = x2 * lax.rsqrt(var + _EPS) * nw_ref[...]
    x2_ref[...] = x2
    h2_ref[...] = h2
    lg_ref[...] = jnp.dot(h2, wr_ref[...], preferred_element_type=jnp.float32)


def _postattn(ao2, x2d, p):
    full = lambda shape: pl.BlockSpec(shape, lambda i: (0,) * len(shape))
    row = lambda w: pl.BlockSpec((_BS, w), lambda i: (i, 0))
    return pl.pallas_call(
        _postattn_body,
        grid=(_S // _BS,),
        in_specs=[row(_D), full((_HQ * _HD, _D)), row(_D), full((1, _D)),
                  full((_D, _E))],
        out_specs=[row(_D), row(_D), row(_E)],
        out_shape=[
            jax.ShapeDtypeStruct((_S, _D), jnp.float32),
            jax.ShapeDtypeStruct((_S, _D), jnp.float32),
            jax.ShapeDtypeStruct((_S, _E), jnp.float32),
        ],
    )(ao2, p['Wo'], x2d, p['ffn_norm_w'].reshape(1, -1), p['Wr'])


def _route_body(lg_ref, dest_ref, wk_ref, aux_ref, counts_ref, psum_ref):
    b = pl.program_id(0)

    @pl.when(b == 0)
    def _():
        counts_ref[...] = jnp.zeros((1, _E), jnp.float32)
        psum_ref[...] = jnp.zeros((1, _E), jnp.float32)

    lg = lg_ref[...]
    m = jnp.max(lg, axis=1, keepdims=True)
    ex = jnp.exp(lg - m)
    prob = ex / jnp.sum(ex, axis=1, keepdims=True)

    @pl.when(b < _S // _BS)
    def _():
        psum_ref[...] += jnp.sum(prob, axis=0, keepdims=True)

    ie = lax.broadcasted_iota(jnp.int32, (_BS, _E), 1)
    m1 = jnp.max(prob, axis=1, keepdims=True)
    i1 = jnp.min(jnp.where(prob >= m1, ie, _E), axis=1, keepdims=True)
    p2 = jnp.where(ie == i1, -1.0, prob)
    m2 = jnp.max(p2, axis=1, keepdims=True)
    i2 = jnp.min(jnp.where(p2 >= m2, ie, _E), axis=1, keepdims=True)
    den = m1 + m2
    c = b // (_S // _BS)
    fe = jnp.where(c == 0, i1, i2)
    w = jnp.where(c == 0, m1, m2) / den
    oh = (ie == fe).astype(jnp.float32)
    ri = lax.broadcasted_iota(jnp.int32, (_BS, _BS), 0)
    ci = lax.broadcasted_iota(jnp.int32, (_BS, _BS), 1)
    ltri = (ci < ri).astype(jnp.float32)
    before = jnp.dot(ltri, oh, preferred_element_type=jnp.float32)
    pos = jnp.sum((counts_ref[...] + before) * oh, axis=1,
                  keepdims=True).astype(jnp.int32)
    counts_ref[...] += jnp.sum(oh, axis=0, keepdims=True)
    keep = pos < _CAP
    posc = jnp.minimum(pos, _CAP - 1)
    entry = b * _BS + lax.broadcasted_iota(jnp.int32, (_BS, 1), 0)
    dest_ref[...] = jnp.where(keep, fe * _CAP + posc, _NSLOT + (entry & (_CAP - 1)))
    wk_ref[...] = jnp.where(keep, w, 0.0)

    @pl.when(b == _NE // _BS - 1)
    def _():
        aux_ref[...] = (_E * jnp.sum(counts_ref[...] * psum_ref[...],
                                     axis=1, keepdims=True)
                        / (float(_NE) * float(_T)))


def _route(logits):
    nb = _NE // _BS
    return pl.pallas_call(
        _route_body,
        grid=(nb,),
        in_specs=[pl.BlockSpec((_BS, _E), lambda b: (b % (_S // _BS), 0))],
        out_specs=[
            pl.BlockSpec((_BS, 1), lambda b: (b, 0)),
            pl.BlockSpec((_BS, 1), lambda b: (b, 0)),
            pl.BlockSpec((1, 1), lambda b: (0, 0)),
        ],
        out_shape=[
            jax.ShapeDtypeStruct((_NE, 1), jnp.int32),
            jax.ShapeDtypeStruct((_NE, 1), jnp.float32),
            jax.ShapeDtypeStruct((1, 1), jnp.float32),
        ],
        scratch_shapes=[pltpu.VMEM((1, _E), jnp.float32),
                        pltpu.VMEM((1, _E), jnp.float32)],
    )(logits)


def _sc_mesh():
    return plsc.VectorSubcoreMesh(core_axis_name="c", subcore_axis_name="s")


def _build_tables(dest, wk):
    per_w = _NSLOT // _NW

    @functools.partial(
        pl.kernel,
        out_type=(jax.ShapeDtypeStruct((_NSLOT,), jnp.int32),
                  jax.ShapeDtypeStruct((_NSLOT,), jnp.float32)),
        mesh=_sc_mesh(),
        scratch_types=[pltpu.VMEM((_NE,), jnp.int32),
                       pltpu.VMEM((_NE,), jnp.float32),
                       pltpu.VMEM((per_w,), jnp.int32),
                       pltpu.VMEM((per_w,), jnp.float32)],
        compiler_params=pltpu.CompilerParams(needs_layout_passes=False),
    )
    def k(dest_hbm, wk_hbm, src_hbm, sw_hbm, dest_v, wk_v, src_l, sw_l):
        wid = lax.axis_index("s") * 2 + lax.axis_index("c")
        lo = wid * per_w
        pltpu.sync_copy(dest_hbm, dest_v)
        pltpu.sync_copy(wk_hbm, wk_v)
        for i in range(per_w // 16):
            evec = lo + i * 16 + jnp.arange(16, dtype=jnp.int32)
            src_l[pl.ds(i * 16, 16)] = evec & (_T - 1)
            sw_l[pl.ds(i * 16, 16)] = jnp.zeros((16,), jnp.float32)

        def body(i, carry):
            d = dest_v[pl.ds(i * 16, 16)]
            w = wk_v[pl.ds(i * 16, 16)]
            evec = i * 16 + jnp.arange(16, dtype=jnp.int32)
            tok = jnp.where(evec >= _T, evec - _T, evec)
            msk = (d >= lo) & (d < lo + per_w)
            plsc.store_scatter(src_l, [d - lo], tok, mask=msk)
            plsc.store_scatter(sw_l, [d - lo], w, mask=msk)
            return carry

        lax.fori_loop(0, _NE // 16, body, 0)
        pltpu.sync_copy(src_l, src_hbm.at[pl.ds(lo, per_w)])
        pltpu.sync_copy(sw_l, sw_hbm.at[pl.ds(lo, per_w)])

    return k(dest, wk)


def _sc_gather(table, idx, n_rows, chunk):
    per_w = n_rows // _NW
    nch = per_w // chunk

    @functools.partial(
        pl.kernel,
        out_type=jax.ShapeDtypeStruct((n_rows, _D), jnp.float32),
        mesh=_sc_mesh(),
        scratch_types=[pltpu.VMEM((chunk,), jnp.int32),
                       pltpu.VMEM((chunk, _D), jnp.float32),
                       pltpu.SemaphoreType.DMA],
        compiler_params=pltpu.CompilerParams(needs_layout_passes=False),
    )
    def k(table_hbm, idx_hbm, out_hbm, idx_v, rows_v, sem):
        wid = lax.axis_index("s") * 2 + lax.axis_index("c")
        base = wid * per_w
        for ci in range(nch):
            off = base + ci * chunk
            pltpu.sync_copy(idx_hbm.at[pl.ds(off, chunk)], idx_v)
            pltpu.async_copy(table_hbm.at[idx_v], rows_v, sem).wait()
            pltpu.sync_copy(rows_v, out_hbm.at[pl.ds(off, chunk)])

    return k(table, idx)


def _expert_body(ein_ref, wg_ref, wu_ref, wd_ref, sw_ref, out_ref):
    e = pl.program_id(0)

    @pl.when(e < _E)
    def _():
        xin = ein_ref[0].astype(jnp.bfloat16)
        g = jnp.dot(xin, wg_ref[0].astype(jnp.bfloat16),
                    preferred_element_type=jnp.float32)
        u = jnp.dot(xin, wu_ref[0].astype(jnp.bfloat16),
                    preferred_element_type=jnp.float32)
        a = (g * lax.logistic(g) * u).astype(jnp.bfloat16)
        o = jnp.dot(a, wd_ref[0].astype(jnp.bfloat16),
                    preferred_element_type=jnp.float32)
        out_ref[0] = o * sw_ref[0]

    @pl.when(e == _E)
    def _():
        out_ref[0] = jnp.zeros((_CAP, _D), jnp.float32)


def _experts(ein3, sw3, p):
    cl = lambda e: (jnp.minimum(e, _E - 1), 0, 0)
    return pl.pallas_call(
        _expert_body,
        grid=(_E + 1,),
        in_specs=[
            pl.BlockSpec((1, _CAP, _D), cl),
            pl.BlockSpec((1, _D, _FF), cl),
            pl.BlockSpec((1, _D, _FF), cl),
            pl.BlockSpec((1, _FF, _D), cl),
            pl.BlockSpec((1, _CAP, 1), cl),
        ],
        out_specs=pl.BlockSpec((1, _CAP, _D), lambda e: (e, 0, 0)),
        out_shape=jax.ShapeDtypeStruct((_E + 1, _CAP, _D), jnp.float32),
    )(ein3, p['Wg'], p['Wu'], p['Wd'], sw3)


def _shared_body(h2_ref, sg_ref, su_ref, sd_ref, o_ref):
    h2 = h2_ref[...].astype(jnp.bfloat16)
    g = jnp.dot(h2, sg_ref[...].astype(jnp.bfloat16),
                preferred_element_type=jnp.float32)
    u = jnp.dot(h2, su_ref[...].astype(jnp.bfloat16),
                preferred_element_type=jnp.float32)
    o_ref[...] = jnp.dot((g * lax.logistic(g) * u).astype(jnp.bfloat16),
                         sd_ref[...].astype(jnp.bfloat16),
                         preferred_element_type=jnp.float32)


def _shared(h2, p):
    full = lambda shape: pl.BlockSpec(shape, lambda i: (0,) * len(shape))
    row = lambda w: pl.BlockSpec((_BS, w), lambda i: (i, 0))
    return pl.pallas_call(
        _shared_body,
        grid=(_S // _BS,),
        in_specs=[row(_D), full((_D, _SFF)), full((_D, _SFF)),
                  full((_SFF, _D))],
        out_specs=row(_D),
        out_shape=jax.ShapeDtypeStruct((_S, _D), jnp.float32),
    )(h2, p['Sg'], p['Su'], p['Sd'])


def _final_body(x2_ref, ta_ref, tb_ref, sh_ref, o_ref):
    o_ref[...] = x2_ref[...] + ta_ref[...] + tb_ref[...] + sh_ref[...]


def _final(x2, tok, shared):
    row = lambda w: pl.BlockSpec((_BS, w), lambda i: (i, 0))
    nb = _S // _BS
    return pl.pallas_call(
        _final_body,
        grid=(nb,),
        in_specs=[
            row(_D),
            pl.BlockSpec((_BS, _D), lambda i: (i, 0)),
            pl.BlockSpec((_BS, _D), lambda i: (i + nb, 0)),
            row(_D),
        ],
        out_specs=row(_D),
        out_shape=jax.ShapeDtypeStruct((_S, _D), jnp.float32),
    )(x2, tok, tok, shared)


def kernel(x, rope_cos, rope_sin, params):
    p = params
    x2d = x.reshape(_S, _D)
    sign = jnp.concatenate([-jnp.ones((_HD // 2,), jnp.float32),
                            jnp.ones((_HD // 2,), jnp.float32)])
    cosq = jnp.tile(rope_cos, (1, _HQ))
    sinq = jnp.tile(rope_sin * sign[None, :], (1, _HQ))
    cosk = jnp.tile(rope_cos, (1, _HKV))
    sink = jnp.tile(rope_sin * sign[None, :], (1, _HKV))

    q, kk, v = _preattn(x2d, p, cosq, sinq, cosk, sink)
    kv_k = kk.reshape(_B, _S, _HKV, _HD)
    kv_v = v.reshape(_B, _S, _HKV, _HD)

    qT = q.reshape(_S, _HQ, _HD).transpose(1, 0, 2)
    kT = kk.reshape(_S, _HKV, _HD).transpose(1, 0, 2)
    vT = v.reshape(_S, _HKV, _HD).transpose(1, 0, 2)
    ao = _attention(qT, kT, vT)
    ao2 = ao.transpose(1, 0, 2).reshape(_S, _HQ * _HD)

    x2, h2, logits = _postattn(ao2, x2d, p)
    shared = _shared(h2, p)
    dest2, wk2, aux = _route(logits)
    dest = dest2.reshape(_NE)
    wk = wk2.reshape(_NE)

    src, sw = _build_tables(dest, wk)

    ein = _sc_gather(h2, src, _NSLOT, 64)
    eout = _experts(ein.reshape(_E, _CAP, _D), sw.reshape(_E, _CAP, 1), p)
    eoutp = eout.reshape((_E + 1) * _CAP, _D)
    tok = _sc_gather(eoutp, dest, _NE, 64)

    out = _final(x2, tok, shared)
    return out.reshape(_B, _S, _D), (kv_k, kv_v), aux.reshape(())


# speedup vs baseline: 1.8274x; 1.0361x over previous
import functools

import jax
import jax.numpy as jnp
from jax import lax
from jax.experimental import pallas as pl
from jax.experimental.pallas import tpu as pltpu
from jax.experimental.pallas import tpu_sc as plsc

_B, _S, _D = 1, 2048, 768
_HQ, _HKV, _HD = 12, 4, 64
_E, _FF, _TOPK, _CAP = 64, 512, 2, 128
_SFF = 2048
_EPS = 1e-6
_T = _B * _S
_NE = _TOPK * _T
_BS = 256
_NSLOT = _E * _CAP
_NW = 32


def _rope(x, cosf, sinf_signed):
    lane = lax.broadcasted_iota(jnp.int32, x.shape, 1)
    left = jnp.roll(x, -32, axis=1)
    right = jnp.roll(x, 32, axis=1)
    swap = jnp.where((lane % _HD) < (_HD // 2), left, right)
    return x * cosf + swap * sinf_signed


def _preattn_body(x_ref, wq_ref, wk_ref, wv_ref, bq_ref, bk_ref, bv_ref,
                  nw_ref, cq_ref, sq_ref, ck_ref, sk_ref,
                  q3_ref, k3_ref, v3_ref, k_ref, v_ref):
    x = x_ref[...]
    var = jnp.mean(x * x, axis=1, keepdims=True)
    h = (x * lax.rsqrt(var + _EPS) * nw_ref[...]).astype(jnp.bfloat16)
    q0 = jnp.dot(h, wq_ref[...].astype(jnp.bfloat16),
                 preferred_element_type=jnp.float32) + bq_ref[...]
    k0 = jnp.dot(h, wk_ref[...].astype(jnp.bfloat16),
                 preferred_element_type=jnp.float32) + bk_ref[...]
    v0 = jnp.dot(h, wv_ref[...].astype(jnp.bfloat16),
                 preferred_element_type=jnp.float32) + bv_ref[...]
    qro = _rope(q0, cq_ref[...], sq_ref[...])
    kro = _rope(k0, ck_ref[...], sk_ref[...])
    for hh in range(_HQ):
        q3_ref[hh] = qro[:, hh * _HD:(hh + 1) * _HD]
    for hh in range(_HKV):
        k3_ref[hh] = kro[:, hh * _HD:(hh + 1) * _HD]
        v3_ref[hh] = v0[:, hh * _HD:(hh + 1) * _HD]
    k_ref[...] = kro
    v_ref[...] = v0


def _preattn(x2d, p, cosq, sinq, cosk, sink):
    full = lambda shape: pl.BlockSpec(shape, lambda i: (0,) * len(shape))
    row = lambda w: pl.BlockSpec((_BS, w), lambda i: (i, 0))
    return pl.pallas_call(
        _preattn_body,
        grid=(_S // _BS,),
        in_specs=[
            row(_D),
            full((_D, _HQ * _HD)), full((_D, _HKV * _HD)), full((_D, _HKV * _HD)),
            full((1, _HQ * _HD)), full((1, _HKV * _HD)), full((1, _HKV * _HD)),
            full((1, _D)),
            row(_HQ * _HD), row(_HQ * _HD), row(_HKV * _HD), row(_HKV * _HD),
        ],
        out_specs=[
            pl.BlockSpec((_HQ, _BS, _HD), lambda i: (0, i, 0)),
            pl.BlockSpec((_HKV, _BS, _HD), lambda i: (0, i, 0)),
            pl.BlockSpec((_HKV, _BS, _HD), lambda i: (0, i, 0)),
            row(_HKV * _HD), row(_HKV * _HD),
        ],
        out_shape=[
            jax.ShapeDtypeStruct((_HQ, _S, _HD), jnp.float32),
            jax.ShapeDtypeStruct((_HKV, _S, _HD), jnp.float32),
            jax.ShapeDtypeStruct((_HKV, _S, _HD), jnp.float32),
            jax.ShapeDtypeStruct((_S, _HKV * _HD), jnp.float32),
            jax.ShapeDtypeStruct((_S, _HKV * _HD), jnp.float32),
        ],
    )(x2d, p['Wq'], p['Wk'], p['Wv'],
      p['bq'].reshape(1, -1), p['bk'].reshape(1, -1), p['bv'].reshape(1, -1),
      p['attn_norm_w'].reshape(1, -1), cosq, sinq, cosk, sink)


def _attn_body(q_ref, k_ref, v_ref, o_ref):
    i = pl.program_id(1)
    q = q_ref[0].astype(jnp.bfloat16)

    def branch(width):
        kb = k_ref[0, 0:width, :].astype(jnp.bfloat16)
        vb = v_ref[0, 0:width, :].astype(jnp.bfloat16)
        s = lax.dot_general(q, kb, (((1,), (1,)), ((), ())),
                            preferred_element_type=jnp.float32) * (1.0 / 8.0)
        rowi = i * _BS + lax.broadcasted_iota(jnp.int32, s.shape, 0)
        coli = lax.broadcasted_iota(jnp.int32, s.shape, 1)
        s = jnp.where(coli <= rowi, s, -1e9)
        m = jnp.max(s, axis=1, keepdims=True)
        e = jnp.exp(s - m)
        a = (e / jnp.sum(e, axis=1, keepdims=True)).astype(jnp.bfloat16)
        o_ref[0] = lax.dot_general(a, vb, (((1,), (0,)), ((), ())),
                                   preferred_element_type=jnp.float32)

    for bi in range(4):
        width = (bi + 1) * 2 * _BS

        @pl.when((i >= 2 * bi) & (i < 2 * bi + 2))
        def _(width=width):
            branch(width)


def _attention(qT, kT, vT):
    rep = _HQ // _HKV
    return pl.pallas_call(
        _attn_body,
        grid=(_HQ, _S // _BS),
        in_specs=[
            pl.BlockSpec((1, _BS, _HD), lambda h, i: (h, i, 0)),
            pl.BlockSpec((1, _S, _HD), lambda h, i: (h // rep, 0, 0)),
            pl.BlockSpec((1, _S, _HD), lambda h, i: (h // rep, 0, 0)),
        ],
        out_specs=pl.BlockSpec((1, _BS, _HD), lambda h, i: (h, i, 0)),
        out_shape=jax.ShapeDtypeStruct((_HQ, _S, _HD), jnp.float32),
    )(qT, kT, vT)


def _postattn_body(ao_ref, wo_ref, x_ref, nw_ref, wr_ref, x2_ref, h2_ref, lg_ref):
    aoc = jnp.concatenate([ao_ref[hh] for hh in range(_HQ)], axis=1)
    x2 = x_ref[...] + jnp.dot(aoc.astype(jnp.bfloat16),
                              wo_ref[...].astype(jnp.bfloat16),
                              preferred_element_type=jnp.float32)
    var = jnp.mean(x2 * x2, axis=1, keepdims=True)
    h2 = x2 * lax.rsqrt(var + _EPS) * nw_ref[...]
    x2_ref[...] = x2
    h2_ref[...] = h2
    lg_ref[...] = jnp.dot(h2, wr_ref[...], preferred_element_type=jnp.float32)


def _postattn(ao2, x2d, p):
    full = lambda shape: pl.BlockSpec(shape, lambda i: (0,) * len(shape))
    row = lambda w: pl.BlockSpec((_BS, w), lambda i: (i, 0))
    return pl.pallas_call(
        _postattn_body,
        grid=(_S // _BS,),
        in_specs=[pl.BlockSpec((_HQ, _BS, _HD), lambda i: (0, i, 0)),
                  full((_HQ * _HD, _D)), row(_D), full((1, _D)),
                  full((_D, _E))],
        out_specs=[row(_D), row(_D), row(_E)],
        out_shape=[
            jax.ShapeDtypeStruct((_S, _D), jnp.float32),
            jax.ShapeDtypeStruct((_S, _D), jnp.float32),
            jax.ShapeDtypeStruct((_S, _E), jnp.float32),
        ],
    )(ao2, p['Wo'], x2d, p['ffn_norm_w'].reshape(1, -1), p['Wr'])


def _route_body(lg_ref, dest_ref, wk_ref, aux_ref, counts_ref, psum_ref):
    b = pl.program_id(0)

    @pl.when(b == 0)
    def _():
        counts_ref[...] = jnp.zeros((1, _E), jnp.float32)
        psum_ref[...] = jnp.zeros((1, _E), jnp.float32)

    lg = lg_ref[...]
    m = jnp.max(lg, axis=1, keepdims=True)
    ex = jnp.exp(lg - m)
    prob = ex / jnp.sum(ex, axis=1, keepdims=True)

    @pl.when(b < _S // _BS)
    def _():
        psum_ref[...] += jnp.sum(prob, axis=0, keepdims=True)

    ie = lax.broadcasted_iota(jnp.int32, (_BS, _E), 1)
    m1 = jnp.max(prob, axis=1, keepdims=True)
    i1 = jnp.min(jnp.where(prob >= m1, ie, _E), axis=1, keepdims=True)
    p2 = jnp.where(ie == i1, -1.0, prob)
    m2 = jnp.max(p2, axis=1, keepdims=True)
    i2 = jnp.min(jnp.where(p2 >= m2, ie, _E), axis=1, keepdims=True)
    den = m1 + m2
    c = b // (_S // _BS)
    fe = jnp.where(c == 0, i1, i2)
    w = jnp.where(c == 0, m1, m2) / den
    oh = (ie == fe).astype(jnp.float32)
    ri = lax.broadcasted_iota(jnp.int32, (_BS, _BS), 0)
    ci = lax.broadcasted_iota(jnp.int32, (_BS, _BS), 1)
    ltri = (ci < ri).astype(jnp.float32)
    before = jnp.dot(ltri, oh, preferred_element_type=jnp.float32)
    pos = jnp.sum((counts_ref[...] + before) * oh, axis=1,
                  keepdims=True).astype(jnp.int32)
    counts_ref[...] += jnp.sum(oh, axis=0, keepdims=True)
    keep = pos < _CAP
    posc = jnp.minimum(pos, _CAP - 1)
    entry = b * _BS + lax.broadcasted_iota(jnp.int32, (_BS, 1), 0)
    dest_ref[...] = jnp.where(keep, fe * _CAP + posc, _NSLOT + (entry & (_CAP - 1)))
    wk_ref[...] = jnp.where(keep, w, 0.0)

    @pl.when(b == _NE // _BS - 1)
    def _():
        aux_ref[...] = (_E * jnp.sum(counts_ref[...] * psum_ref[...],
                                     axis=1, keepdims=True)
                        / (float(_NE) * float(_T)))


def _route(logits):
    nb = _NE // _BS
    return pl.pallas_call(
        _route_body,
        grid=(nb,),
        in_specs=[pl.BlockSpec((_BS, _E), lambda b: (b % (_S // _BS), 0))],
        out_specs=[
            pl.BlockSpec((_BS, 1), lambda b: (b, 0)),
            pl.BlockSpec((_BS, 1), lambda b: (b, 0)),
            pl.BlockSpec((1, 1), lambda b: (0, 0)),
        ],
        out_shape=[
            jax.ShapeDtypeStruct((_NE, 1), jnp.int32),
            jax.ShapeDtypeStruct((_NE, 1), jnp.float32),
            jax.ShapeDtypeStruct((1, 1), jnp.float32),
        ],
        scratch_shapes=[pltpu.VMEM((1, _E), jnp.float32),
                        pltpu.VMEM((1, _E), jnp.float32)],
    )(logits)


def _sc_mesh():
    return plsc.VectorSubcoreMesh(core_axis_name="c", subcore_axis_name="s")


def _build_tables(dest, wk):
    per_w = _NSLOT // _NW

    @functools.partial(
        pl.kernel,
        out_type=(jax.ShapeDtypeStruct((_NSLOT,), jnp.int32),
                  jax.ShapeDtypeStruct((_NSLOT,), jnp.float32)),
        mesh=_sc_mesh(),
        scratch_types=[pltpu.VMEM((_NE,), jnp.int32),
                       pltpu.VMEM((_NE,), jnp.float32),
                       pltpu.VMEM((per_w,), jnp.int32),
                       pltpu.VMEM((per_w,), jnp.float32)],
        compiler_params=pltpu.CompilerParams(needs_layout_passes=False),
    )
    def k(dest_hbm, wk_hbm, src_hbm, sw_hbm, dest_v, wk_v, src_l, sw_l):
        wid = lax.axis_index("s") * 2 + lax.axis_index("c")
        lo = wid * per_w
        pltpu.sync_copy(dest_hbm, dest_v)
        pltpu.sync_copy(wk_hbm, wk_v)
        for i in range(per_w // 16):
            evec = lo + i * 16 + jnp.arange(16, dtype=jnp.int32)
            src_l[pl.ds(i * 16, 16)] = evec & (_T - 1)
            sw_l[pl.ds(i * 16, 16)] = jnp.zeros((16,), jnp.float32)

        def body(i, carry):
            d = dest_v[pl.ds(i * 16, 16)]
            w = wk_v[pl.ds(i * 16, 16)]
            evec = i * 16 + jnp.arange(16, dtype=jnp.int32)
            tok = jnp.where(evec >= _T, evec - _T, evec)
            msk = (d >= lo) & (d < lo + per_w)
            plsc.store_scatter(src_l, [d - lo], tok, mask=msk)
            plsc.store_scatter(sw_l, [d - lo], w, mask=msk)
            return carry

        lax.fori_loop(0, _NE // 16, body, 0)
        pltpu.sync_copy(src_l, src_hbm.at[pl.ds(lo, per_w)])
        pltpu.sync_copy(sw_l, sw_hbm.at[pl.ds(lo, per_w)])

    return k(dest, wk)


def _sc_gather(table, idx, n_rows, chunk):
    per_w = n_rows // _NW
    nch = per_w // chunk

    @functools.partial(
        pl.kernel,
        out_type=jax.ShapeDtypeStruct((n_rows, _D), jnp.float32),
        mesh=_sc_mesh(),
        scratch_types=[pltpu.VMEM((chunk,), jnp.int32),
                       pltpu.VMEM((chunk, _D), jnp.float32),
                       pltpu.SemaphoreType.DMA],
        compiler_params=pltpu.CompilerParams(needs_layout_passes=False),
    )
    def k(table_hbm, idx_hbm, out_hbm, idx_v, rows_v, sem):
        wid = lax.axis_index("s") * 2 + lax.axis_index("c")
        base = wid * per_w
        for ci in range(nch):
            off = base + ci * chunk
            pltpu.sync_copy(idx_hbm.at[pl.ds(off, chunk)], idx_v)
            pltpu.async_copy(table_hbm.at[idx_v], rows_v, sem).wait()
            pltpu.sync_copy(rows_v, out_hbm.at[pl.ds(off, chunk)])

    return k(table, idx)


def _expert_body(ein_ref, wg_ref, wu_ref, wd_ref, sw_ref, out_ref):
    e = pl.program_id(0)

    @pl.when(e < _E)
    def _():
        xin = ein_ref[0].astype(jnp.bfloat16)
        g = jnp.dot(xin, wg_ref[0].astype(jnp.bfloat16),
                    preferred_element_type=jnp.float32)
        u = jnp.dot(xin, wu_ref[0].astype(jnp.bfloat16),
                    preferred_element_type=jnp.float32)
        a = (g * lax.logistic(g) * u).astype(jnp.bfloat16)
        o = jnp.dot(a, wd_ref[0].astype(jnp.bfloat16),
                    preferred_element_type=jnp.float32)
        out_ref[0] = o * sw_ref[0]

    @pl.when(e == _E)
    def _():
        out_ref[0] = jnp.zeros((_CAP, _D), jnp.float32)


def _experts(ein3, sw3, p):
    cl = lambda e: (jnp.minimum(e, _E - 1), 0, 0)
    return pl.pallas_call(
        _expert_body,
        grid=(_E + 1,),
        in_specs=[
            pl.BlockSpec((1, _CAP, _D), cl),
            pl.BlockSpec((1, _D, _FF), cl),
            pl.BlockSpec((1, _D, _FF), cl),
            pl.BlockSpec((1, _FF, _D), cl),
            pl.BlockSpec((1, _CAP, 1), cl),
        ],
        out_specs=pl.BlockSpec((1, _CAP, _D), lambda e: (e, 0, 0)),
        out_shape=jax.ShapeDtypeStruct((_E + 1, _CAP, _D), jnp.float32),
    )(ein3, p['Wg'], p['Wu'], p['Wd'], sw3)


def _shared_body(h2_ref, sg_ref, su_ref, sd_ref, o_ref):
    h2 = h2_ref[...].astype(jnp.bfloat16)
    g = jnp.dot(h2, sg_ref[...].astype(jnp.bfloat16),
                preferred_element_type=jnp.float32)
    u = jnp.dot(h2, su_ref[...].astype(jnp.bfloat16),
                preferred_element_type=jnp.float32)
    o_ref[...] = jnp.dot((g * lax.logistic(g) * u).astype(jnp.bfloat16),
                         sd_ref[...].astype(jnp.bfloat16),
                         preferred_element_type=jnp.float32)


def _shared(h2, p):
    full = lambda shape: pl.BlockSpec(shape, lambda i: (0,) * len(shape))
    row = lambda w: pl.BlockSpec((_BS, w), lambda i: (i, 0))
    return pl.pallas_call(
        _shared_body,
        grid=(_S // _BS,),
        in_specs=[row(_D), full((_D, _SFF)), full((_D, _SFF)),
                  full((_SFF, _D))],
        out_specs=row(_D),
        out_shape=jax.ShapeDtypeStruct((_S, _D), jnp.float32),
    )(h2, p['Sg'], p['Su'], p['Sd'])


def _final_body(x2_ref, ta_ref, tb_ref, sh_ref, o_ref):
    o_ref[...] = x2_ref[...] + ta_ref[...] + tb_ref[...] + sh_ref[...]


def _final(x2, tok, shared):
    row = lambda w: pl.BlockSpec((_BS, w), lambda i: (i, 0))
    nb = _S // _BS
    return pl.pallas_call(
        _final_body,
        grid=(nb,),
        in_specs=[
            row(_D),
            pl.BlockSpec((_BS, _D), lambda i: (i, 0)),
            pl.BlockSpec((_BS, _D), lambda i: (i + nb, 0)),
            row(_D),
        ],
        out_specs=row(_D),
        out_shape=jax.ShapeDtypeStruct((_S, _D), jnp.float32),
    )(x2, tok, tok, shared)


def kernel(x, rope_cos, rope_sin, params):
    p = params
    x2d = x.reshape(_S, _D)
    sign = jnp.concatenate([-jnp.ones((_HD // 2,), jnp.float32),
                            jnp.ones((_HD // 2,), jnp.float32)])
    cosq = jnp.tile(rope_cos, (1, _HQ))
    sinq = jnp.tile(rope_sin * sign[None, :], (1, _HQ))
    cosk = jnp.tile(rope_cos, (1, _HKV))
    sink = jnp.tile(rope_sin * sign[None, :], (1, _HKV))

    qT, kT, vT, kk, v = _preattn(x2d, p, cosq, sinq, cosk, sink)
    kv_k = kk.reshape(_B, _S, _HKV, _HD)
    kv_v = v.reshape(_B, _S, _HKV, _HD)

    ao = _attention(qT, kT, vT)

    x2, h2, logits = _postattn(ao, x2d, p)
    shared = _shared(h2, p)
    dest2, wk2, aux = _route(logits)
    dest = dest2.reshape(_NE)
    wk = wk2.reshape(_NE)

    src, sw = _build_tables(dest, wk)

    ein = _sc_gather(h2, src, _NSLOT, 64)
    eout = _experts(ein.reshape(_E, _CAP, _D), sw.reshape(_E, _CAP, 1), p)
    eoutp = eout.reshape((_E + 1) * _CAP, _D)
    tok = _sc_gather(eoutp, dest, _NE, 64)

    out = _final(x2, tok, shared)
    return out.reshape(_B, _S, _D), (kv_k, kv_v), aux.reshape(())


# speedup vs baseline: 1.8328x; 1.0029x over previous
import functools

import jax
import jax.numpy as jnp
from jax import lax
from jax.experimental import pallas as pl
from jax.experimental.pallas import tpu as pltpu
from jax.experimental.pallas import tpu_sc as plsc

_B, _S, _D = 1, 2048, 768
_HQ, _HKV, _HD = 12, 4, 64
_E, _FF, _TOPK, _CAP = 64, 512, 2, 128
_SFF = 2048
_EPS = 1e-6
_T = _B * _S
_NE = _TOPK * _T
_BS = 256
_NSLOT = _E * _CAP
_NW = 32


def _rope(x, cosf, sinf_signed):
    lane = lax.broadcasted_iota(jnp.int32, x.shape, 1)
    left = jnp.roll(x, -32, axis=1)
    right = jnp.roll(x, 32, axis=1)
    swap = jnp.where((lane % _HD) < (_HD // 2), left, right)
    return x * cosf + swap * sinf_signed


def _preattn_body(x_ref, wq_ref, wk_ref, wv_ref, bq_ref, bk_ref, bv_ref,
                  nw_ref, cq_ref, sq_ref, ck_ref, sk_ref,
                  q3_ref, k3_ref, v3_ref, k_ref, v_ref):
    x = x_ref[...]
    var = jnp.mean(x * x, axis=1, keepdims=True)
    h = (x * lax.rsqrt(var + _EPS) * nw_ref[...]).astype(jnp.bfloat16)
    q0 = jnp.dot(h, wq_ref[...].astype(jnp.bfloat16),
                 preferred_element_type=jnp.float32) + bq_ref[...]
    k0 = jnp.dot(h, wk_ref[...].astype(jnp.bfloat16),
                 preferred_element_type=jnp.float32) + bk_ref[...]
    v0 = jnp.dot(h, wv_ref[...].astype(jnp.bfloat16),
                 preferred_element_type=jnp.float32) + bv_ref[...]
    qro = _rope(q0, cq_ref[...], sq_ref[...])
    kro = _rope(k0, ck_ref[...], sk_ref[...])
    for hh in range(_HQ):
        q3_ref[hh] = qro[:, hh * _HD:(hh + 1) * _HD]
    for hh in range(_HKV):
        k3_ref[hh] = kro[:, hh * _HD:(hh + 1) * _HD]
        v3_ref[hh] = v0[:, hh * _HD:(hh + 1) * _HD]
    k_ref[...] = kro
    v_ref[...] = v0


def _preattn(x2d, p, cosq, sinq, cosk, sink):
    full = lambda shape: pl.BlockSpec(shape, lambda i: (0,) * len(shape))
    row = lambda w: pl.BlockSpec((_BS, w), lambda i: (i, 0))
    return pl.pallas_call(
        _preattn_body,
        grid=(_S // _BS,),
        in_specs=[
            row(_D),
            full((_D, _HQ * _HD)), full((_D, _HKV * _HD)), full((_D, _HKV * _HD)),
            full((1, _HQ * _HD)), full((1, _HKV * _HD)), full((1, _HKV * _HD)),
            full((1, _D)),
            row(_HQ * _HD), row(_HQ * _HD), row(_HKV * _HD), row(_HKV * _HD),
        ],
        out_specs=[
            pl.BlockSpec((_HQ, _BS, _HD), lambda i: (0, i, 0)),
            pl.BlockSpec((_HKV, _BS, _HD), lambda i: (0, i, 0)),
            pl.BlockSpec((_HKV, _BS, _HD), lambda i: (0, i, 0)),
            row(_HKV * _HD), row(_HKV * _HD),
        ],
        out_shape=[
            jax.ShapeDtypeStruct((_HQ, _S, _HD), jnp.float32),
            jax.ShapeDtypeStruct((_HKV, _S, _HD), jnp.float32),
            jax.ShapeDtypeStruct((_HKV, _S, _HD), jnp.float32),
            jax.ShapeDtypeStruct((_S, _HKV * _HD), jnp.float32),
            jax.ShapeDtypeStruct((_S, _HKV * _HD), jnp.float32),
        ],
    )(x2d, p['Wq'], p['Wk'], p['Wv'],
      p['bq'].reshape(1, -1), p['bk'].reshape(1, -1), p['bv'].reshape(1, -1),
      p['attn_norm_w'].reshape(1, -1), cosq, sinq, cosk, sink)


def _attn_body(q_ref, k_ref, v_ref, o_ref):
    i = pl.program_id(1)
    q = q_ref[0].astype(jnp.bfloat16)

    def branch(width):
        kb = k_ref[0, 0:width, :].astype(jnp.bfloat16)
        vb = v_ref[0, 0:width, :].astype(jnp.bfloat16)
        s = lax.dot_general(q, kb, (((1,), (1,)), ((), ())),
                            preferred_element_type=jnp.float32) * (1.0 / 8.0)
        rowi = i * _BS + lax.broadcasted_iota(jnp.int32, s.shape, 0)
        coli = lax.broadcasted_iota(jnp.int32, s.shape, 1)
        s = jnp.where(coli <= rowi, s, -1e9)
        m = jnp.max(s, axis=1, keepdims=True)
        e = jnp.exp(s - m)
        a = (e / jnp.sum(e, axis=1, keepdims=True)).astype(jnp.bfloat16)
        o_ref[0] = lax.dot_general(a, vb, (((1,), (0,)), ((), ())),
                                   preferred_element_type=jnp.float32)

    for bi in range(4):
        width = (bi + 1) * 2 * _BS

        @pl.when((i >= 2 * bi) & (i < 2 * bi + 2))
        def _(width=width):
            branch(width)


def _attention(qT, kT, vT):
    rep = _HQ // _HKV
    return pl.pallas_call(
        _attn_body,
        grid=(_HQ, _S // _BS),
        in_specs=[
            pl.BlockSpec((1, _BS, _HD), lambda h, i: (h, i, 0)),
            pl.BlockSpec((1, _S, _HD), lambda h, i: (h // rep, 0, 0)),
            pl.BlockSpec((1, _S, _HD), lambda h, i: (h // rep, 0, 0)),
        ],
        out_specs=pl.BlockSpec((1, _BS, _HD), lambda h, i: (h, i, 0)),
        out_shape=jax.ShapeDtypeStruct((_HQ, _S, _HD), jnp.float32),
    )(qT, kT, vT)


def _postattn_body(ao_ref, wo_ref, x_ref, nw_ref, wr_ref, x2_ref, h2_ref, lg_ref):
    aoc = jnp.concatenate([ao_ref[hh] for hh in range(_HQ)], axis=1)
    x2 = x_ref[...] + jnp.dot(aoc.astype(jnp.bfloat16),
                              wo_ref[...].astype(jnp.bfloat16),
                              preferred_element_type=jnp.float32)
    var = jnp.mean(x2 * x2, axis=1, keepdims=True)
    h2 = x2 * lax.rsqrt(var + _EPS) * nw_ref[...]
    x2_ref[...] = x2
    h2_ref[...] = h2
    lg_ref[...] = jnp.dot(h2, wr_ref[...], preferred_element_type=jnp.float32)


def _postattn(ao2, x2d, p):
    full = lambda shape: pl.BlockSpec(shape, lambda i: (0,) * len(shape))
    row = lambda w: pl.BlockSpec((_BS, w), lambda i: (i, 0))
    return pl.pallas_call(
        _postattn_body,
        grid=(_S // _BS,),
        in_specs=[pl.BlockSpec((_HQ, _BS, _HD), lambda i: (0, i, 0)),
                  full((_HQ * _HD, _D)), row(_D), full((1, _D)),
                  full((_D, _E))],
        out_specs=[row(_D), row(_D), row(_E)],
        out_shape=[
            jax.ShapeDtypeStruct((_S, _D), jnp.float32),
            jax.ShapeDtypeStruct((_S, _D), jnp.float32),
            jax.ShapeDtypeStruct((_S, _E), jnp.float32),
        ],
    )(ao2, p['Wo'], x2d, p['ffn_norm_w'].reshape(1, -1), p['Wr'])


def _route_body(lg_ref, dest_ref, wk_ref, aux_ref, counts_ref, psum_ref):
    b = pl.program_id(0)

    @pl.when(b == 0)
    def _():
        counts_ref[...] = jnp.zeros((1, _E), jnp.float32)
        psum_ref[...] = jnp.zeros((1, _E), jnp.float32)

    lg = lg_ref[...]
    m = jnp.max(lg, axis=1, keepdims=True)
    ex = jnp.exp(lg - m)
    prob = ex / jnp.sum(ex, axis=1, keepdims=True)

    @pl.when(b < _S // _BS)
    def _():
        psum_ref[...] += jnp.sum(prob, axis=0, keepdims=True)

    ie = lax.broadcasted_iota(jnp.int32, (_BS, _E), 1)
    m1 = jnp.max(prob, axis=1, keepdims=True)
    i1 = jnp.min(jnp.where(prob >= m1, ie, _E), axis=1, keepdims=True)
    p2 = jnp.where(ie == i1, -1.0, prob)
    m2 = jnp.max(p2, axis=1, keepdims=True)
    i2 = jnp.min(jnp.where(p2 >= m2, ie, _E), axis=1, keepdims=True)
    den = m1 + m2
    c = b // (_S // _BS)
    fe = jnp.where(c == 0, i1, i2)
    w = jnp.where(c == 0, m1, m2) / den
    oh = (ie == fe).astype(jnp.float32)
    ri = lax.broadcasted_iota(jnp.int32, (_BS, _BS), 0)
    ci = lax.broadcasted_iota(jnp.int32, (_BS, _BS), 1)
    ltri = (ci < ri).astype(jnp.bfloat16)
    before = jnp.dot(ltri, oh.astype(jnp.bfloat16),
                     preferred_element_type=jnp.float32)
    pos = jnp.sum((counts_ref[...] + before) * oh, axis=1,
                  keepdims=True).astype(jnp.int32)
    counts_ref[...] += jnp.sum(oh, axis=0, keepdims=True)
    keep = pos < _CAP
    posc = jnp.minimum(pos, _CAP - 1)
    entry = b * _BS + lax.broadcasted_iota(jnp.int32, (_BS, 1), 0)
    dest_ref[...] = jnp.where(keep, fe * _CAP + posc, _NSLOT + (entry & (_CAP - 1)))
    wk_ref[...] = jnp.where(keep, w, 0.0)

    @pl.when(b == _NE // _BS - 1)
    def _():
        aux_ref[...] = (_E * jnp.sum(counts_ref[...] * psum_ref[...],
                                     axis=1, keepdims=True)
                        / (float(_NE) * float(_T)))


def _route(logits):
    nb = _NE // _BS
    return pl.pallas_call(
        _route_body,
        grid=(nb,),
        in_specs=[pl.BlockSpec((_BS, _E), lambda b: (b % (_S // _BS), 0))],
        out_specs=[
            pl.BlockSpec((_BS, 1), lambda b: (b, 0)),
            pl.BlockSpec((_BS, 1), lambda b: (b, 0)),
            pl.BlockSpec((1, 1), lambda b: (0, 0)),
        ],
        out_shape=[
            jax.ShapeDtypeStruct((_NE, 1), jnp.int32),
            jax.ShapeDtypeStruct((_NE, 1), jnp.float32),
            jax.ShapeDtypeStruct((1, 1), jnp.float32),
        ],
        scratch_shapes=[pltpu.VMEM((1, _E), jnp.float32),
                        pltpu.VMEM((1, _E), jnp.float32)],
    )(logits)


def _sc_mesh():
    return plsc.VectorSubcoreMesh(core_axis_name="c", subcore_axis_name="s")


def _build_tables(dest, wk):
    per_w = _NSLOT // _NW

    @functools.partial(
        pl.kernel,
        out_type=(jax.ShapeDtypeStruct((_NSLOT,), jnp.int32),
                  jax.ShapeDtypeStruct((_NSLOT,), jnp.float32)),
        mesh=_sc_mesh(),
        scratch_types=[pltpu.VMEM((_NE,), jnp.int32),
                       pltpu.VMEM((_NE,), jnp.float32),
                       pltpu.VMEM((per_w,), jnp.int32),
                       pltpu.VMEM((per_w,), jnp.float32)],
        compiler_params=pltpu.CompilerParams(needs_layout_passes=False),
    )
    def k(dest_hbm, wk_hbm, src_hbm, sw_hbm, dest_v, wk_v, src_l, sw_l):
        wid = lax.axis_index("s") * 2 + lax.axis_index("c")
        lo = wid * per_w
        pltpu.sync_copy(dest_hbm, dest_v)
        pltpu.sync_copy(wk_hbm, wk_v)
        for i in range(per_w // 16):
            evec = lo + i * 16 + jnp.arange(16, dtype=jnp.int32)
            src_l[pl.ds(i * 16, 16)] = evec & (_T - 1)
            sw_l[pl.ds(i * 16, 16)] = jnp.zeros((16,), jnp.float32)

        def body(i, carry):
            d = dest_v[pl.ds(i * 16, 16)]
            w = wk_v[pl.ds(i * 16, 16)]
            evec = i * 16 + jnp.arange(16, dtype=jnp.int32)
            tok = jnp.where(evec >= _T, evec - _T, evec)
            msk = (d >= lo) & (d < lo + per_w)
            plsc.store_scatter(src_l, [d - lo], tok, mask=msk)
            plsc.store_scatter(sw_l, [d - lo], w, mask=msk)
            return carry

        lax.fori_loop(0, _NE // 16, body, 0)
        pltpu.sync_copy(src_l, src_hbm.at[pl.ds(lo, per_w)])
        pltpu.sync_copy(sw_l, sw_hbm.at[pl.ds(lo, per_w)])

    return k(dest, wk)


def _sc_gather(table, idx, n_rows, chunk):
    per_w = n_rows // _NW
    nch = per_w // chunk

    @functools.partial(
        pl.kernel,
        out_type=jax.ShapeDtypeStruct((n_rows, _D), jnp.float32),
        mesh=_sc_mesh(),
        scratch_types=[pltpu.VMEM((chunk,), jnp.int32),
                       pltpu.VMEM((chunk,), jnp.int32),
                       pltpu.VMEM((chunk, _D), jnp.float32),
                       pltpu.VMEM((chunk, _D), jnp.float32),
                       pltpu.SemaphoreType.DMA,
                       pltpu.SemaphoreType.DMA],
        compiler_params=pltpu.CompilerParams(needs_layout_passes=False),
    )
    def k(table_hbm, idx_hbm, out_hbm, ia, ib, ra, rb, sa, sb):
        wid = lax.axis_index("s") * 2 + lax.axis_index("c")
        base = wid * per_w
        bufs = [(ia, ra, sa), (ib, rb, sb)]
        copies = [None] * nch
        for ci in range(nch):
            iv, rv, sem = bufs[ci % 2]
            pltpu.sync_copy(idx_hbm.at[pl.ds(base + ci * chunk, chunk)], iv)
            copies[ci] = pltpu.async_copy(table_hbm.at[iv], rv, sem)
            if ci >= 1:
                pv = bufs[(ci - 1) % 2][1]
                copies[ci - 1].wait()
                pltpu.sync_copy(
                    pv, out_hbm.at[pl.ds(base + (ci - 1) * chunk, chunk)])
        copies[nch - 1].wait()
        pltpu.sync_copy(
            bufs[(nch - 1) % 2][1],
            out_hbm.at[pl.ds(base + (nch - 1) * chunk, chunk)])

    return k(table, idx)


def _expert_body(ein_ref, wg_ref, wu_ref, wd_ref, sw_ref, out_ref):
    e = pl.program_id(0)

    @pl.when(e < _E)
    def _():
        xin = ein_ref[0].astype(jnp.bfloat16)
        g = jnp.dot(xin, wg_ref[0].astype(jnp.bfloat16),
                    preferred_element_type=jnp.float32)
        u = jnp.dot(xin, wu_ref[0].astype(jnp.bfloat16),
                    preferred_element_type=jnp.float32)
        a = (g * lax.logistic(g) * u).astype(jnp.bfloat16)
        o = jnp.dot(a, wd_ref[0].astype(jnp.bfloat16),
                    preferred_element_type=jnp.float32)
        out_ref[0] = o * sw_ref[0]

    @pl.when(e == _E)
    def _():
        out_ref[0] = jnp.zeros((_CAP, _D), jnp.float32)


def _experts(ein3, sw3, p):
    cl = lambda e: (jnp.minimum(e, _E - 1), 0, 0)
    return pl.pallas_call(
        _expert_body,
        grid=(_E + 1,),
        in_specs=[
            pl.BlockSpec((1, _CAP, _D), cl),
            pl.BlockSpec((1, _D, _FF), cl),
            pl.BlockSpec((1, _D, _FF), cl),
            pl.BlockSpec((1, _FF, _D), cl),
            pl.BlockSpec((1, _CAP, 1), cl),
        ],
        out_specs=pl.BlockSpec((1, _CAP, _D), lambda e: (e, 0, 0)),
        out_shape=jax.ShapeDtypeStruct((_E + 1, _CAP, _D), jnp.float32),
    )(ein3, p['Wg'], p['Wu'], p['Wd'], sw3)


def _shared_body(h2_ref, sg_ref, su_ref, sd_ref, o_ref):
    h2 = h2_ref[...].astype(jnp.bfloat16)
    g = jnp.dot(h2, sg_ref[...].astype(jnp.bfloat16),
                preferred_element_type=jnp.float32)
    u = jnp.dot(h2, su_ref[...].astype(jnp.bfloat16),
                preferred_element_type=jnp.float32)
    o_ref[...] = jnp.dot((g * lax.logistic(g) * u).astype(jnp.bfloat16),
                         sd_ref[...].astype(jnp.bfloat16),
                         preferred_element_type=jnp.float32)


def _shared(h2, p):
    full = lambda shape: pl.BlockSpec(shape, lambda i: (0,) * len(shape))
    row = lambda w: pl.BlockSpec((_BS, w), lambda i: (i, 0))
    return pl.pallas_call(
        _shared_body,
        grid=(_S // _BS,),
        in_specs=[row(_D), full((_D, _SFF)), full((_D, _SFF)),
                  full((_SFF, _D))],
        out_specs=row(_D),
        out_shape=jax.ShapeDtypeStruct((_S, _D), jnp.float32),
    )(h2, p['Sg'], p['Su'], p['Sd'])


def _final_body(x2_ref, ta_ref, tb_ref, sh_ref, o_ref):
    o_ref[...] = x2_ref[...] + ta_ref[...] + tb_ref[...] + sh_ref[...]


def _final(x2, tok, shared):
    row = lambda w: pl.BlockSpec((_BS, w), lambda i: (i, 0))
    nb = _S // _BS
    return pl.pallas_call(
        _final_body,
        grid=(nb,),
        in_specs=[
            row(_D),
            pl.BlockSpec((_BS, _D), lambda i: (i, 0)),
            pl.BlockSpec((_BS, _D), lambda i: (i + nb, 0)),
            row(_D),
        ],
        out_specs=row(_D),
        out_shape=jax.ShapeDtypeStruct((_S, _D), jnp.float32),
    )(x2, tok, tok, shared)


def kernel(x, rope_cos, rope_sin, params):
    p = params
    x2d = x.reshape(_S, _D)
    sign = jnp.concatenate([-jnp.ones((_HD // 2,), jnp.float32),
                            jnp.ones((_HD // 2,), jnp.float32)])
    cosq = jnp.tile(rope_cos, (1, _HQ))
    sinq = jnp.tile(rope_sin * sign[None, :], (1, _HQ))
    cosk = jnp.tile(rope_cos, (1, _HKV))
    sink = jnp.tile(rope_sin * sign[None, :], (1, _HKV))

    qT, kT, vT, kk, v = _preattn(x2d, p, cosq, sinq, cosk, sink)
    kv_k = kk.reshape(_B, _S, _HKV, _HD)
    kv_v = v.reshape(_B, _S, _HKV, _HD)

    ao = _attention(qT, kT, vT)

    x2, h2, logits = _postattn(ao, x2d, p)
    shared = _shared(h2, p)
    dest2, wk2, aux = _route(logits)
    dest = dest2.reshape(_NE)
    wk = wk2.reshape(_NE)

    src, sw = _build_tables(dest, wk)

    ein = _sc_gather(h2, src, _NSLOT, 64)
    eout = _experts(ein.reshape(_E, _CAP, _D), sw.reshape(_E, _CAP, 1), p)
    eoutp = eout.reshape((_E + 1) * _CAP, _D)
    tok = _sc_gather(eoutp, dest, _NE, 64)

    out = _final(x2, tok, shared)
    return out.reshape(_B, _S, _D), (kv_k, kv_v), aux.reshape(())


# speedup vs baseline: 1.8423x; 1.0052x over previous
import functools

import jax
import jax.numpy as jnp
from jax import lax
from jax.experimental import pallas as pl
from jax.experimental.pallas import tpu as pltpu
from jax.experimental.pallas import tpu_sc as plsc

_B, _S, _D = 1, 2048, 768
_HQ, _HKV, _HD = 12, 4, 64
_E, _FF, _TOPK, _CAP = 64, 512, 2, 128
_SFF = 2048
_EPS = 1e-6
_T = _B * _S
_NE = _TOPK * _T
_BS = 256
_NSLOT = _E * _CAP
_NW = 32


def _rope(x, cosf, sinf_signed):
    lane = lax.broadcasted_iota(jnp.int32, x.shape, 1)
    left = jnp.roll(x, -32, axis=1)
    right = jnp.roll(x, 32, axis=1)
    swap = jnp.where((lane % _HD) < (_HD // 2), left, right)
    return x * cosf + swap * sinf_signed


def _preattn_body(x_ref, wq_ref, wk_ref, wv_ref, bq_ref, bk_ref, bv_ref,
                  nw_ref, cq_ref, sq_ref, ck_ref, sk_ref,
                  q3_ref, k3_ref, v3_ref, k_ref, v_ref):
    x = x_ref[...]
    var = jnp.mean(x * x, axis=1, keepdims=True)
    h = (x * lax.rsqrt(var + _EPS) * nw_ref[...]).astype(jnp.bfloat16)
    q0 = jnp.dot(h, wq_ref[...].astype(jnp.bfloat16),
                 preferred_element_type=jnp.float32) + bq_ref[...]
    k0 = jnp.dot(h, wk_ref[...].astype(jnp.bfloat16),
                 preferred_element_type=jnp.float32) + bk_ref[...]
    v0 = jnp.dot(h, wv_ref[...].astype(jnp.bfloat16),
                 preferred_element_type=jnp.float32) + bv_ref[...]
    qro = _rope(q0, cq_ref[...], sq_ref[...])
    kro = _rope(k0, ck_ref[...], sk_ref[...])
    for hh in range(_HQ):
        q3_ref[hh] = qro[:, hh * _HD:(hh + 1) * _HD]
    for hh in range(_HKV):
        k3_ref[hh] = kro[:, hh * _HD:(hh + 1) * _HD]
        v3_ref[hh] = v0[:, hh * _HD:(hh + 1) * _HD]
    k_ref[...] = kro
    v_ref[...] = v0


def _preattn(x2d, p, cosq, sinq, cosk, sink):
    full = lambda shape: pl.BlockSpec(shape, lambda i: (0,) * len(shape))
    row = lambda w: pl.BlockSpec((_BS, w), lambda i: (i, 0))
    return pl.pallas_call(
        _preattn_body,
        grid=(_S // _BS,),
        in_specs=[
            row(_D),
            full((_D, _HQ * _HD)), full((_D, _HKV * _HD)), full((_D, _HKV * _HD)),
            full((1, _HQ * _HD)), full((1, _HKV * _HD)), full((1, _HKV * _HD)),
            full((1, _D)),
            row(_HQ * _HD), row(_HQ * _HD), row(_HKV * _HD), row(_HKV * _HD),
        ],
        out_specs=[
            pl.BlockSpec((_HQ, _BS, _HD), lambda i: (0, i, 0)),
            pl.BlockSpec((_HKV, _BS, _HD), lambda i: (0, i, 0)),
            pl.BlockSpec((_HKV, _BS, _HD), lambda i: (0, i, 0)),
            row(_HKV * _HD), row(_HKV * _HD),
        ],
        out_shape=[
            jax.ShapeDtypeStruct((_HQ, _S, _HD), jnp.float32),
            jax.ShapeDtypeStruct((_HKV, _S, _HD), jnp.float32),
            jax.ShapeDtypeStruct((_HKV, _S, _HD), jnp.float32),
            jax.ShapeDtypeStruct((_S, _HKV * _HD), jnp.float32),
            jax.ShapeDtypeStruct((_S, _HKV * _HD), jnp.float32),
        ],
    )(x2d, p['Wq'], p['Wk'], p['Wv'],
      p['bq'].reshape(1, -1), p['bk'].reshape(1, -1), p['bv'].reshape(1, -1),
      p['attn_norm_w'].reshape(1, -1), cosq, sinq, cosk, sink)


_AQ = 512


def _attn_body(q_ref, k_ref, v_ref, o_ref):
    i = pl.program_id(1)
    q = q_ref[0].astype(jnp.bfloat16)

    def branch(width):
        kb = k_ref[0, 0:width, :].astype(jnp.bfloat16)
        vb = v_ref[0, 0:width, :].astype(jnp.bfloat16)
        s = lax.dot_general(q, kb, (((1,), (1,)), ((), ())),
                            preferred_element_type=jnp.float32) * (1.0 / 8.0)
        rowi = i * _AQ + lax.broadcasted_iota(jnp.int32, s.shape, 0)
        coli = lax.broadcasted_iota(jnp.int32, s.shape, 1)
        s = jnp.where(coli <= rowi, s, -1e9)
        m = jnp.max(s, axis=1, keepdims=True)
        e = jnp.exp(s - m)
        a = (e / jnp.sum(e, axis=1, keepdims=True)).astype(jnp.bfloat16)
        o_ref[0] = lax.dot_general(a, vb, (((1,), (0,)), ((), ())),
                                   preferred_element_type=jnp.float32)

    for bi in range(_S // _AQ):
        @pl.when(i == bi)
        def _(width=(bi + 1) * _AQ):
            branch(width)


def _attention(qT, kT, vT):
    rep = _HQ // _HKV
    return pl.pallas_call(
        _attn_body,
        grid=(_HQ, _S // _AQ),
        in_specs=[
            pl.BlockSpec((1, _AQ, _HD), lambda h, i: (h, i, 0)),
            pl.BlockSpec((1, _S, _HD), lambda h, i: (h // rep, 0, 0)),
            pl.BlockSpec((1, _S, _HD), lambda h, i: (h // rep, 0, 0)),
        ],
        out_specs=pl.BlockSpec((1, _AQ, _HD), lambda h, i: (h, i, 0)),
        out_shape=jax.ShapeDtypeStruct((_HQ, _S, _HD), jnp.float32),
    )(qT, kT, vT)


def _postattn_body(ao_ref, wo_ref, x_ref, nw_ref, wr_ref, x2_ref, h2_ref, lg_ref):
    aoc = jnp.concatenate([ao_ref[hh] for hh in range(_HQ)], axis=1)
    x2 = x_ref[...] + jnp.dot(aoc.astype(jnp.bfloat16),
                              wo_ref[...].astype(jnp.bfloat16),
                              preferred_element_type=jnp.float32)
    var = jnp.mean(x2 * x2, axis=1, keepdims=True)
    h2 = x2 * lax.rsqrt(var + _EPS) * nw_ref[...]
    x2_ref[...] = x2
    h2_ref[...] = h2
    lg_ref[...] = jnp.dot(h2, wr_ref[...], preferred_element_type=jnp.float32)


def _postattn(ao2, x2d, p):
    full = lambda shape: pl.BlockSpec(shape, lambda i: (0,) * len(shape))
    row = lambda w: pl.BlockSpec((_BS, w), lambda i: (i, 0))
    return pl.pallas_call(
        _postattn_body,
        grid=(_S // _BS,),
        in_specs=[pl.BlockSpec((_HQ, _BS, _HD), lambda i: (0, i, 0)),
                  full((_HQ * _HD, _D)), row(_D), full((1, _D)),
                  full((_D, _E))],
        out_specs=[row(_D), row(_D), row(_E)],
        out_shape=[
            jax.ShapeDtypeStruct((_S, _D), jnp.float32),
            jax.ShapeDtypeStruct((_S, _D), jnp.float32),
            jax.ShapeDtypeStruct((_S, _E), jnp.float32),
        ],
    )(ao2, p['Wo'], x2d, p['ffn_norm_w'].reshape(1, -1), p['Wr'])


def _route_body(lg_ref, dest_ref, wk_ref, aux_ref, counts_ref, psum_ref):
    b = pl.program_id(0)

    @pl.when(b == 0)
    def _():
        counts_ref[...] = jnp.zeros((1, _E), jnp.float32)
        psum_ref[...] = jnp.zeros((1, _E), jnp.float32)

    lg = lg_ref[...]
    m = jnp.max(lg, axis=1, keepdims=True)
    ex = jnp.exp(lg - m)
    prob = ex / jnp.sum(ex, axis=1, keepdims=True)

    @pl.when(b < _S // _BS)
    def _():
        psum_ref[...] += jnp.sum(prob, axis=0, keepdims=True)

    ie = lax.broadcasted_iota(jnp.int32, (_BS, _E), 1)
    m1 = jnp.max(prob, axis=1, keepdims=True)
    i1 = jnp.min(jnp.where(prob >= m1, ie, _E), axis=1, keepdims=True)
    p2 = jnp.where(ie == i1, -1.0, prob)
    m2 = jnp.max(p2, axis=1, keepdims=True)
    i2 = jnp.min(jnp.where(p2 >= m2, ie, _E), axis=1, keepdims=True)
    den = m1 + m2
    c = b // (_S // _BS)
    fe = jnp.where(c == 0, i1, i2)
    w = jnp.where(c == 0, m1, m2) / den
    oh = (ie == fe).astype(jnp.float32)
    ri = lax.broadcasted_iota(jnp.int32, (_BS, _BS), 0)
    ci = lax.broadcasted_iota(jnp.int32, (_BS, _BS), 1)
    ltri = (ci < ri).astype(jnp.bfloat16)
    before = jnp.dot(ltri, oh.astype(jnp.bfloat16),
                     preferred_element_type=jnp.float32)
    pos = jnp.sum((counts_ref[...] + before) * oh, axis=1,
                  keepdims=True).astype(jnp.int32)
    counts_ref[...] += jnp.sum(oh, axis=0, keepdims=True)
    keep = pos < _CAP
    posc = jnp.minimum(pos, _CAP - 1)
    entry = b * _BS + lax.broadcasted_iota(jnp.int32, (_BS, 1), 0)
    dest_ref[...] = jnp.where(keep, fe * _CAP + posc, _NSLOT + (entry & (_CAP - 1)))
    wk_ref[...] = jnp.where(keep, w, 0.0)

    @pl.when(b == _NE // _BS - 1)
    def _():
        aux_ref[...] = (_E * jnp.sum(counts_ref[...] * psum_ref[...],
                                     axis=1, keepdims=True)
                        / (float(_NE) * float(_T)))


def _route(logits):
    nb = _NE // _BS
    return pl.pallas_call(
        _route_body,
        grid=(nb,),
        in_specs=[pl.BlockSpec((_BS, _E), lambda b: (b % (_S // _BS), 0))],
        out_specs=[
            pl.BlockSpec((_BS, 1), lambda b: (b, 0)),
            pl.BlockSpec((_BS, 1), lambda b: (b, 0)),
            pl.BlockSpec((1, 1), lambda b: (0, 0)),
        ],
        out_shape=[
            jax.ShapeDtypeStruct((_NE, 1), jnp.int32),
            jax.ShapeDtypeStruct((_NE, 1), jnp.float32),
            jax.ShapeDtypeStruct((1, 1), jnp.float32),
        ],
        scratch_shapes=[pltpu.VMEM((1, _E), jnp.float32),
                        pltpu.VMEM((1, _E), jnp.float32)],
    )(logits)


def _sc_mesh():
    return plsc.VectorSubcoreMesh(core_axis_name="c", subcore_axis_name="s")


def _build_tables(dest, wk):
    per_w = _NSLOT // _NW

    @functools.partial(
        pl.kernel,
        out_type=(jax.ShapeDtypeStruct((_NSLOT,), jnp.int32),
                  jax.ShapeDtypeStruct((_NSLOT,), jnp.float32)),
        mesh=_sc_mesh(),
        scratch_types=[pltpu.VMEM((_NE,), jnp.int32),
                       pltpu.VMEM((_NE,), jnp.float32),
                       pltpu.VMEM((per_w,), jnp.int32),
                       pltpu.VMEM((per_w,), jnp.float32)],
        compiler_params=pltpu.CompilerParams(needs_layout_passes=False),
    )
    def k(dest_hbm, wk_hbm, src_hbm, sw_hbm, dest_v, wk_v, src_l, sw_l):
        wid = lax.axis_index("s") * 2 + lax.axis_index("c")
        lo = wid * per_w
        pltpu.sync_copy(dest_hbm, dest_v)
        pltpu.sync_copy(wk_hbm, wk_v)
        for i in range(per_w // 16):
            evec = lo + i * 16 + jnp.arange(16, dtype=jnp.int32)
            src_l[pl.ds(i * 16, 16)] = evec & (_T - 1)
            sw_l[pl.ds(i * 16, 16)] = jnp.zeros((16,), jnp.float32)

        def body(i, carry):
            d = dest_v[pl.ds(i * 16, 16)]
            w = wk_v[pl.ds(i * 16, 16)]
            evec = i * 16 + jnp.arange(16, dtype=jnp.int32)
            tok = jnp.where(evec >= _T, evec - _T, evec)
            msk = (d >= lo) & (d < lo + per_w)
            plsc.store_scatter(src_l, [d - lo], tok, mask=msk)
            plsc.store_scatter(sw_l, [d - lo], w, mask=msk)
            return carry

        lax.fori_loop(0, _NE // 16, body, 0)
        pltpu.sync_copy(src_l, src_hbm.at[pl.ds(lo, per_w)])
        pltpu.sync_copy(sw_l, sw_hbm.at[pl.ds(lo, per_w)])

    return k(dest, wk)


def _sc_gather(table, idx, n_rows, chunk):
    per_w = n_rows // _NW
    nch = per_w // chunk

    @functools.partial(
        pl.kernel,
        out_type=jax.ShapeDtypeStruct((n_rows, _D), jnp.float32),
        mesh=_sc_mesh(),
        scratch_types=[pltpu.VMEM((chunk,), jnp.int32),
                       pltpu.VMEM((chunk,), jnp.int32),
                       pltpu.VMEM((chunk, _D), jnp.float32),
                       pltpu.VMEM((chunk, _D), jnp.float32),
                       pltpu.SemaphoreType.DMA,
                       pltpu.SemaphoreType.DMA],
        compiler_params=pltpu.CompilerParams(needs_layout_passes=False),
    )
    def k(table_hbm, idx_hbm, out_hbm, ia, ib, ra, rb, sa, sb):
        wid = lax.axis_index("s") * 2 + lax.axis_index("c")
        base = wid * per_w
        bufs = [(ia, ra, sa), (ib, rb, sb)]
        copies = [None] * nch
        for ci in range(nch):
            iv, rv, sem = bufs[ci % 2]
            pltpu.sync_copy(idx_hbm.at[pl.ds(base + ci * chunk, chunk)], iv)
            copies[ci] = pltpu.async_copy(table_hbm.at[iv], rv, sem)
            if ci >= 1:
                pv = bufs[(ci - 1) % 2][1]
                copies[ci - 1].wait()
                pltpu.sync_copy(
                    pv, out_hbm.at[pl.ds(base + (ci - 1) * chunk, chunk)])
        copies[nch - 1].wait()
        pltpu.sync_copy(
            bufs[(nch - 1) % 2][1],
            out_hbm.at[pl.ds(base + (nch - 1) * chunk, chunk)])

    return k(table, idx)


def _expert_body(ein_ref, wg_ref, wu_ref, wd_ref, sw_ref, out_ref):
    e = pl.program_id(0)

    @pl.when(e < _E)
    def _():
        xin = ein_ref[0].astype(jnp.bfloat16)
        g = jnp.dot(xin, wg_ref[0].astype(jnp.bfloat16),
                    preferred_element_type=jnp.float32)
        u = jnp.dot(xin, wu_ref[0].astype(jnp.bfloat16),
                    preferred_element_type=jnp.float32)
        a = (g * lax.logistic(g) * u).astype(jnp.bfloat16)
        o = jnp.dot(a, wd_ref[0].astype(jnp.bfloat16),
                    preferred_element_type=jnp.float32)
        out_ref[0] = o * sw_ref[0]

    @pl.when(e == _E)
    def _():
        out_ref[0] = jnp.zeros((_CAP, _D), jnp.float32)


def _experts(ein3, sw3, p):
    cl = lambda e: (jnp.minimum(e, _E - 1), 0, 0)
    return pl.pallas_call(
        _expert_body,
        grid=(_E + 1,),
        in_specs=[
            pl.BlockSpec((1, _CAP, _D), cl),
            pl.BlockSpec((1, _D, _FF), cl),
            pl.BlockSpec((1, _D, _FF), cl),
            pl.BlockSpec((1, _FF, _D), cl),
            pl.BlockSpec((1, _CAP, 1), cl),
        ],
        out_specs=pl.BlockSpec((1, _CAP, _D), lambda e: (e, 0, 0)),
        out_shape=jax.ShapeDtypeStruct((_E + 1, _CAP, _D), jnp.float32),
    )(ein3, p['Wg'], p['Wu'], p['Wd'], sw3)


def _shared_body(h2_ref, sg_ref, su_ref, sd_ref, o_ref):
    h2 = h2_ref[...].astype(jnp.bfloat16)
    g = jnp.dot(h2, sg_ref[...].astype(jnp.bfloat16),
                preferred_element_type=jnp.float32)
    u = jnp.dot(h2, su_ref[...].astype(jnp.bfloat16),
                preferred_element_type=jnp.float32)
    o_ref[...] = jnp.dot((g * lax.logistic(g) * u).astype(jnp.bfloat16),
                         sd_ref[...].astype(jnp.bfloat16),
                         preferred_element_type=jnp.float32)


def _shared(h2, p):
    full = lambda shape: pl.BlockSpec(shape, lambda i: (0,) * len(shape))
    row = lambda w: pl.BlockSpec((_BS, w), lambda i: (i, 0))
    return pl.pallas_call(
        _shared_body,
        grid=(_S // _BS,),
        in_specs=[row(_D), full((_D, _SFF)), full((_D, _SFF)),
                  full((_SFF, _D))],
        out_specs=row(_D),
        out_shape=jax.ShapeDtypeStruct((_S, _D), jnp.float32),
    )(h2, p['Sg'], p['Su'], p['Sd'])


def _final_body(x2_ref, ta_ref, tb_ref, sh_ref, o_ref):
    o_ref[...] = x2_ref[...] + ta_ref[...] + tb_ref[...] + sh_ref[...]


def _final(x2, tok, shared):
    row = lambda w: pl.BlockSpec((_BS, w), lambda i: (i, 0))
    nb = _S // _BS
    return pl.pallas_call(
        _final_body,
        grid=(nb,),
        in_specs=[
            row(_D),
            pl.BlockSpec((_BS, _D), lambda i: (i, 0)),
            pl.BlockSpec((_BS, _D), lambda i: (i + nb, 0)),
            row(_D),
        ],
        out_specs=row(_D),
        out_shape=jax.ShapeDtypeStruct((_S, _D), jnp.float32),
    )(x2, tok, tok, shared)


def kernel(x, rope_cos, rope_sin, params):
    p = params
    x2d = x.reshape(_S, _D)
    sign = jnp.concatenate([-jnp.ones((_HD // 2,), jnp.float32),
                            jnp.ones((_HD // 2,), jnp.float32)])
    cosq = jnp.tile(rope_cos, (1, _HQ))
    sinq = jnp.tile(rope_sin * sign[None, :], (1, _HQ))
    cosk = jnp.tile(rope_cos, (1, _HKV))
    sink = jnp.tile(rope_sin * sign[None, :], (1, _HKV))

    qT, kT, vT, kk, v = _preattn(x2d, p, cosq, sinq, cosk, sink)
    kv_k = kk.reshape(_B, _S, _HKV, _HD)
    kv_v = v.reshape(_B, _S, _HKV, _HD)

    ao = _attention(qT, kT, vT)

    x2, h2, logits = _postattn(ao, x2d, p)
    shared = _shared(h2, p)
    dest2, wk2, aux = _route(logits)
    dest = dest2.reshape(_NE)
    wk = wk2.reshape(_NE)

    src, sw = _build_tables(dest, wk)

    ein = _sc_gather(h2, src, _NSLOT, 64)
    eout = _experts(ein.reshape(_E, _CAP, _D), sw.reshape(_E, _CAP, 1), p)
    eoutp = eout.reshape((_E + 1) * _CAP, _D)
    tok = _sc_gather(eoutp, dest, _NE, 64)

    out = _final(x2, tok, shared)
    return out.reshape(_B, _S, _D), (kv_k, kv_v), aux.reshape(())


# speedup vs baseline: 1.9595x; 1.0636x over previous
import functools

import jax
import jax.numpy as jnp
from jax import lax
from jax.experimental import pallas as pl
from jax.experimental.pallas import tpu as pltpu
from jax.experimental.pallas import tpu_sc as plsc

_B, _S, _D = 1, 2048, 768
_HQ, _HKV, _HD = 12, 4, 64
_E, _FF, _TOPK, _CAP = 64, 512, 2, 128
_SFF = 2048
_EPS = 1e-6
_T = _B * _S
_NE = _TOPK * _T
_BS = 256
_NSLOT = _E * _CAP
_NW = 32


def _rope(x, cosf, sinf_signed):
    lane = lax.broadcasted_iota(jnp.int32, x.shape, 1)
    left = jnp.roll(x, -32, axis=1)
    right = jnp.roll(x, 32, axis=1)
    swap = jnp.where((lane % _HD) < (_HD // 2), left, right)
    return x * cosf + swap * sinf_signed


def _preattn_body(x_ref, wq_ref, wk_ref, wv_ref, bq_ref, bk_ref, bv_ref,
                  nw_ref, cq_ref, sq_ref,
                  q3_ref, k3_ref, v3_ref, k_ref, v_ref):
    x = x_ref[...]
    var = jnp.mean(x * x, axis=1, keepdims=True)
    h = (x * lax.rsqrt(var + _EPS) * nw_ref[...]).astype(jnp.bfloat16)
    q0 = jnp.dot(h, wq_ref[...].astype(jnp.bfloat16),
                 preferred_element_type=jnp.float32) + bq_ref[...]
    k0 = jnp.dot(h, wk_ref[...].astype(jnp.bfloat16),
                 preferred_element_type=jnp.float32) + bk_ref[...]
    v0 = jnp.dot(h, wv_ref[...].astype(jnp.bfloat16),
                 preferred_element_type=jnp.float32) + bv_ref[...]
    cos4 = jnp.concatenate([cq_ref[...]] * _HKV, axis=1)
    sin4 = jnp.concatenate([sq_ref[...]] * _HKV, axis=1)
    cos12 = jnp.concatenate([cos4] * (_HQ // _HKV), axis=1)
    sin12 = jnp.concatenate([sin4] * (_HQ // _HKV), axis=1)
    qro = _rope(q0, cos12, sin12)
    kro = _rope(k0, cos4, sin4)
    for hh in range(_HQ):
        q3_ref[hh] = qro[:, hh * _HD:(hh + 1) * _HD]
    for hh in range(_HKV):
        k3_ref[hh] = kro[:, hh * _HD:(hh + 1) * _HD]
        v3_ref[hh] = v0[:, hh * _HD:(hh + 1) * _HD]
    k_ref[...] = kro
    v_ref[...] = v0


def _preattn(x2d, p, cosq, sinq):
    full = lambda shape: pl.BlockSpec(shape, lambda i: (0,) * len(shape))
    row = lambda w: pl.BlockSpec((_BS, w), lambda i: (i, 0))
    return pl.pallas_call(
        _preattn_body,
        grid=(_S // _BS,),
        in_specs=[
            row(_D),
            full((_D, _HQ * _HD)), full((_D, _HKV * _HD)), full((_D, _HKV * _HD)),
            full((1, _HQ * _HD)), full((1, _HKV * _HD)), full((1, _HKV * _HD)),
            full((1, _D)),
            row(_HD), row(_HD),
        ],
        out_specs=[
            pl.BlockSpec((_HQ, _BS, _HD), lambda i: (0, i, 0)),
            pl.BlockSpec((_HKV, _BS, _HD), lambda i: (0, i, 0)),
            pl.BlockSpec((_HKV, _BS, _HD), lambda i: (0, i, 0)),
            row(_HKV * _HD), row(_HKV * _HD),
        ],
        out_shape=[
            jax.ShapeDtypeStruct((_HQ, _S, _HD), jnp.float32),
            jax.ShapeDtypeStruct((_HKV, _S, _HD), jnp.float32),
            jax.ShapeDtypeStruct((_HKV, _S, _HD), jnp.float32),
            jax.ShapeDtypeStruct((_S, _HKV * _HD), jnp.float32),
            jax.ShapeDtypeStruct((_S, _HKV * _HD), jnp.float32),
        ],
    )(x2d, p['Wq'], p['Wk'], p['Wv'],
      p['bq'].reshape(1, -1), p['bk'].reshape(1, -1), p['bv'].reshape(1, -1),
      p['attn_norm_w'].reshape(1, -1), cosq, sinq)


_AQ = 512


def _attn_body(q_ref, k_ref, v_ref, o_ref):
    i = pl.program_id(1)
    q = q_ref[0].astype(jnp.bfloat16)

    def branch(width):
        kb = k_ref[0, 0:width, :].astype(jnp.bfloat16)
        vb = v_ref[0, 0:width, :].astype(jnp.bfloat16)
        s = lax.dot_general(q, kb, (((1,), (1,)), ((), ())),
                            preferred_element_type=jnp.float32) * (1.0 / 8.0)
        rowi = i * _AQ + lax.broadcasted_iota(jnp.int32, s.shape, 0)
        coli = lax.broadcasted_iota(jnp.int32, s.shape, 1)
        s = jnp.where(coli <= rowi, s, -1e9)
        m = jnp.max(s, axis=1, keepdims=True)
        e = jnp.exp(s - m)
        a = (e / jnp.sum(e, axis=1, keepdims=True)).astype(jnp.bfloat16)
        o_ref[0] = lax.dot_general(a, vb, (((1,), (0,)), ((), ())),
                                   preferred_element_type=jnp.float32)

    for bi in range(_S // _AQ):
        @pl.when(i == bi)
        def _(width=(bi + 1) * _AQ):
            branch(width)


def _attention(qT, kT, vT):
    rep = _HQ // _HKV
    return pl.pallas_call(
        _attn_body,
        grid=(_HQ, _S // _AQ),
        in_specs=[
            pl.BlockSpec((1, _AQ, _HD), lambda h, i: (h, i, 0)),
            pl.BlockSpec((1, _S, _HD), lambda h, i: (h // rep, 0, 0)),
            pl.BlockSpec((1, _S, _HD), lambda h, i: (h // rep, 0, 0)),
        ],
        out_specs=pl.BlockSpec((1, _AQ, _HD), lambda h, i: (h, i, 0)),
        out_shape=jax.ShapeDtypeStruct((_HQ, _S, _HD), jnp.float32),
    )(qT, kT, vT)


def _postattn_body(ao_ref, wo_ref, x_ref, nw_ref, wr_ref, x2_ref, h2_ref, lg_ref):
    aoc = jnp.concatenate([ao_ref[hh] for hh in range(_HQ)], axis=1)
    x2 = x_ref[...] + jnp.dot(aoc.astype(jnp.bfloat16),
                              wo_ref[...].astype(jnp.bfloat16),
                              preferred_element_type=jnp.float32)
    var = jnp.mean(x2 * x2, axis=1, keepdims=True)
    h2 = x2 * lax.rsqrt(var + _EPS) * nw_ref[...]
    x2_ref[...] = x2
    h2_ref[...] = h2
    lg_ref[...] = jnp.dot(h2, wr_ref[...], preferred_element_type=jnp.float32)


def _postattn(ao2, x2d, p):
    full = lambda shape: pl.BlockSpec(shape, lambda i: (0,) * len(shape))
    row = lambda w: pl.BlockSpec((_BS, w), lambda i: (i, 0))
    return pl.pallas_call(
        _postattn_body,
        grid=(_S // _BS,),
        in_specs=[pl.BlockSpec((_HQ, _BS, _HD), lambda i: (0, i, 0)),
                  full((_HQ * _HD, _D)), row(_D), full((1, _D)),
                  full((_D, _E))],
        out_specs=[row(_D), row(_D), row(_E)],
        out_shape=[
            jax.ShapeDtypeStruct((_S, _D), jnp.float32),
            jax.ShapeDtypeStruct((_S, _D), jnp.float32),
            jax.ShapeDtypeStruct((_S, _E), jnp.float32),
        ],
    )(ao2, p['Wo'], x2d, p['ffn_norm_w'].reshape(1, -1), p['Wr'])


def _route_body(lg_ref, dest_ref, wk_ref, aux_ref, counts_ref, psum_ref):
    b = pl.program_id(0)

    @pl.when(b == 0)
    def _():
        counts_ref[...] = jnp.zeros((1, _E), jnp.float32)
        psum_ref[...] = jnp.zeros((1, _E), jnp.float32)

    lg = lg_ref[...]
    m = jnp.max(lg, axis=1, keepdims=True)
    ex = jnp.exp(lg - m)
    prob = ex / jnp.sum(ex, axis=1, keepdims=True)

    @pl.when(b < _S // _BS)
    def _():
        psum_ref[...] += jnp.sum(prob, axis=0, keepdims=True)

    ie = lax.broadcasted_iota(jnp.int32, (_BS, _E), 1)
    m1 = jnp.max(prob, axis=1, keepdims=True)
    i1 = jnp.min(jnp.where(prob >= m1, ie, _E), axis=1, keepdims=True)
    p2 = jnp.where(ie == i1, -1.0, prob)
    m2 = jnp.max(p2, axis=1, keepdims=True)
    i2 = jnp.min(jnp.where(p2 >= m2, ie, _E), axis=1, keepdims=True)
    den = m1 + m2
    c = b // (_S // _BS)
    fe = jnp.where(c == 0, i1, i2)
    w = jnp.where(c == 0, m1, m2) / den
    oh = (ie == fe).astype(jnp.float32)
    ri = lax.broadcasted_iota(jnp.int32, (_BS, _BS), 0)
    ci = lax.broadcasted_iota(jnp.int32, (_BS, _BS), 1)
    ltri = (ci < ri).astype(jnp.bfloat16)
    before = jnp.dot(ltri, oh.astype(jnp.bfloat16),
                     preferred_element_type=jnp.float32)
    pos = jnp.sum((counts_ref[...] + before) * oh, axis=1,
                  keepdims=True).astype(jnp.int32)
    counts_ref[...] += jnp.sum(oh, axis=0, keepdims=True)
    keep = pos < _CAP
    posc = jnp.minimum(pos, _CAP - 1)
    entry = b * _BS + lax.broadcasted_iota(jnp.int32, (_BS, 1), 0)
    dest_ref[...] = jnp.where(keep, fe * _CAP + posc, _NSLOT + (entry & (_CAP - 1)))
    wk_ref[...] = jnp.where(keep, w, 0.0)

    @pl.when(b == _NE // _BS - 1)
    def _():
        aux_ref[...] = (_E * jnp.sum(counts_ref[...] * psum_ref[...],
                                     axis=1, keepdims=True)
                        / (float(_NE) * float(_T)))


def _route(logits):
    nb = _NE // _BS
    return pl.pallas_call(
        _route_body,
        grid=(nb,),
        in_specs=[pl.BlockSpec((_BS, _E), lambda b: (b % (_S // _BS), 0))],
        out_specs=[
            pl.BlockSpec((_BS, 1), lambda b: (b, 0)),
            pl.BlockSpec((_BS, 1), lambda b: (b, 0)),
            pl.BlockSpec((1, 1), lambda b: (0, 0)),
        ],
        out_shape=[
            jax.ShapeDtypeStruct((_NE, 1), jnp.int32),
            jax.ShapeDtypeStruct((_NE, 1), jnp.float32),
            jax.ShapeDtypeStruct((1, 1), jnp.float32),
        ],
        scratch_shapes=[pltpu.VMEM((1, _E), jnp.float32),
                        pltpu.VMEM((1, _E), jnp.float32)],
    )(logits)


def _sc_mesh():
    return plsc.VectorSubcoreMesh(core_axis_name="c", subcore_axis_name="s")


def _build_tables(dest, wk):
    per_w = _NSLOT // _NW

    @functools.partial(
        pl.kernel,
        out_type=(jax.ShapeDtypeStruct((_NSLOT,), jnp.int32),
                  jax.ShapeDtypeStruct((_NSLOT,), jnp.float32)),
        mesh=_sc_mesh(),
        scratch_types=[pltpu.VMEM((_NE,), jnp.int32),
                       pltpu.VMEM((_NE,), jnp.float32),
                       pltpu.VMEM((per_w,), jnp.int32),
                       pltpu.VMEM((per_w,), jnp.float32)],
        compiler_params=pltpu.CompilerParams(needs_layout_passes=False),
    )
    def k(dest_hbm, wk_hbm, src_hbm, sw_hbm, dest_v, wk_v, src_l, sw_l):
        wid = lax.axis_index("s") * 2 + lax.axis_index("c")
        lo = wid * per_w
        pltpu.sync_copy(dest_hbm, dest_v)
        pltpu.sync_copy(wk_hbm, wk_v)
        for i in range(per_w // 16):
            evec = lo + i * 16 + jnp.arange(16, dtype=jnp.int32)
            src_l[pl.ds(i * 16, 16)] = evec & (_T - 1)
            sw_l[pl.ds(i * 16, 16)] = jnp.zeros((16,), jnp.float32)

        def body(i, carry):
            d = dest_v[pl.ds(i * 16, 16)]
            w = wk_v[pl.ds(i * 16, 16)]
            evec = i * 16 + jnp.arange(16, dtype=jnp.int32)
            tok = jnp.where(evec >= _T, evec - _T, evec)
            msk = (d >= lo) & (d < lo + per_w)
            plsc.store_scatter(src_l, [d - lo], tok, mask=msk)
            plsc.store_scatter(sw_l, [d - lo], w, mask=msk)
            return carry

        lax.fori_loop(0, _NE // 16, body, 0)
        pltpu.sync_copy(src_l, src_hbm.at[pl.ds(lo, per_w)])
        pltpu.sync_copy(sw_l, sw_hbm.at[pl.ds(lo, per_w)])

    return k(dest, wk)


def _sc_gather(table, idx, n_rows, chunk):
    per_w = n_rows // _NW
    nch = per_w // chunk

    @functools.partial(
        pl.kernel,
        out_type=jax.ShapeDtypeStruct((n_rows, _D), jnp.float32),
        mesh=_sc_mesh(),
        scratch_types=[pltpu.VMEM((chunk,), jnp.int32),
                       pltpu.VMEM((chunk,), jnp.int32),
                       pltpu.VMEM((chunk, _D), jnp.float32),
                       pltpu.VMEM((chunk, _D), jnp.float32),
                       pltpu.SemaphoreType.DMA,
                       pltpu.SemaphoreType.DMA],
        compiler_params=pltpu.CompilerParams(needs_layout_passes=False),
    )
    def k(table_hbm, idx_hbm, out_hbm, ia, ib, ra, rb, sa, sb):
        wid = lax.axis_index("s") * 2 + lax.axis_index("c")
        base = wid * per_w
        bufs = [(ia, ra, sa), (ib, rb, sb)]
        copies = [None] * nch
        for ci in range(nch):
            iv, rv, sem = bufs[ci % 2]
            pltpu.sync_copy(idx_hbm.at[pl.ds(base + ci * chunk, chunk)], iv)
            copies[ci] = pltpu.async_copy(table_hbm.at[iv], rv, sem)
            if ci >= 1:
                pv = bufs[(ci - 1) % 2][1]
                copies[ci - 1].wait()
                pltpu.sync_copy(
                    pv, out_hbm.at[pl.ds(base + (ci - 1) * chunk, chunk)])
        copies[nch - 1].wait()
        pltpu.sync_copy(
            bufs[(nch - 1) % 2][1],
            out_hbm.at[pl.ds(base + (nch - 1) * chunk, chunk)])

    return k(table, idx)


def _expert_body(ein_ref, wg_ref, wu_ref, wd_ref, sw_ref, out_ref):
    e = pl.program_id(0)

    @pl.when(e < _E)
    def _():
        xin = ein_ref[0].astype(jnp.bfloat16)
        g = jnp.dot(xin, wg_ref[0].astype(jnp.bfloat16),
                    preferred_element_type=jnp.float32)
        u = jnp.dot(xin, wu_ref[0].astype(jnp.bfloat16),
                    preferred_element_type=jnp.float32)
        a = (g * lax.logistic(g) * u).astype(jnp.bfloat16)
        o = jnp.dot(a, wd_ref[0].astype(jnp.bfloat16),
                    preferred_element_type=jnp.float32)
        out_ref[0] = o * sw_ref[0]

    @pl.when(e == _E)
    def _():
        out_ref[0] = jnp.zeros((_CAP, _D), jnp.float32)


def _experts(ein3, sw3, p):
    cl = lambda e: (jnp.minimum(e, _E - 1), 0, 0)
    return pl.pallas_call(
        _expert_body,
        grid=(_E + 1,),
        in_specs=[
            pl.BlockSpec((1, _CAP, _D), cl),
            pl.BlockSpec((1, _D, _FF), cl),
            pl.BlockSpec((1, _D, _FF), cl),
            pl.BlockSpec((1, _FF, _D), cl),
            pl.BlockSpec((1, _CAP, 1), cl),
        ],
        out_specs=pl.BlockSpec((1, _CAP, _D), lambda e: (e, 0, 0)),
        out_shape=jax.ShapeDtypeStruct((_E + 1, _CAP, _D), jnp.float32),
    )(ein3, p['Wg'], p['Wu'], p['Wd'], sw3)


def _shared_body(h2_ref, sg_ref, su_ref, sd_ref, o_ref):
    h2 = h2_ref[...].astype(jnp.bfloat16)
    g = jnp.dot(h2, sg_ref[...].astype(jnp.bfloat16),
                preferred_element_type=jnp.float32)
    u = jnp.dot(h2, su_ref[...].astype(jnp.bfloat16),
                preferred_element_type=jnp.float32)
    o_ref[...] = jnp.dot((g * lax.logistic(g) * u).astype(jnp.bfloat16),
                         sd_ref[...].astype(jnp.bfloat16),
                         preferred_element_type=jnp.float32)


def _shared(h2, p):
    full = lambda shape: pl.BlockSpec(shape, lambda i: (0,) * len(shape))
    row = lambda w: pl.BlockSpec((_BS, w), lambda i: (i, 0))
    return pl.pallas_call(
        _shared_body,
        grid=(_S // _BS,),
        in_specs=[row(_D), full((_D, _SFF)), full((_D, _SFF)),
                  full((_SFF, _D))],
        out_specs=row(_D),
        out_shape=jax.ShapeDtypeStruct((_S, _D), jnp.float32),
    )(h2, p['Sg'], p['Su'], p['Sd'])


def _final_body(x2_ref, ta_ref, tb_ref, sh_ref, o_ref):
    o_ref[...] = x2_ref[...] + ta_ref[...] + tb_ref[...] + sh_ref[...]


def _final(x2, tok, shared):
    row = lambda w: pl.BlockSpec((_BS, w), lambda i: (i, 0))
    nb = _S // _BS
    return pl.pallas_call(
        _final_body,
        grid=(nb,),
        in_specs=[
            row(_D),
            pl.BlockSpec((_BS, _D), lambda i: (i, 0)),
            pl.BlockSpec((_BS, _D), lambda i: (i + nb, 0)),
            row(_D),
        ],
        out_specs=row(_D),
        out_shape=jax.ShapeDtypeStruct((_S, _D), jnp.float32),
    )(x2, tok, tok, shared)


def kernel(x, rope_cos, rope_sin, params):
    p = params
    x2d = x.reshape(_S, _D)
    sign = jnp.concatenate([-jnp.ones((_HD // 2,), jnp.float32),
                            jnp.ones((_HD // 2,), jnp.float32)])
    cosq = rope_cos
    sinq = rope_sin * sign[None, :]

    qT, kT, vT, kk, v = _preattn(x2d, p, cosq, sinq)
    kv_k = kk.reshape(_B, _S, _HKV, _HD)
    kv_v = v.reshape(_B, _S, _HKV, _HD)

    ao = _attention(qT, kT, vT)

    x2, h2, logits = _postattn(ao, x2d, p)
    shared = _shared(h2, p)
    dest2, wk2, aux = _route(logits)
    dest = dest2.reshape(_NE)
    wk = wk2.reshape(_NE)

    src, sw = _build_tables(dest, wk)

    ein = _sc_gather(h2, src, _NSLOT, 64)
    eout = _experts(ein.reshape(_E, _CAP, _D), sw.reshape(_E, _CAP, 1), p)
    eoutp = eout.reshape((_E + 1) * _CAP, _D)
    tok = _sc_gather(eoutp, dest, _NE, 64)

    out = _final(x2, tok, shared)
    return out.reshape(_B, _S, _D), (kv_k, kv_v), aux.reshape(())


# speedup vs baseline: 2.0022x; 1.0218x over previous
import functools

import jax
import jax.numpy as jnp
from jax import lax
from jax.experimental import pallas as pl
from jax.experimental.pallas import tpu as pltpu
from jax.experimental.pallas import tpu_sc as plsc

_B, _S, _D = 1, 2048, 768
_HQ, _HKV, _HD = 12, 4, 64
_E, _FF, _TOPK, _CAP = 64, 512, 2, 128
_SFF = 2048
_EPS = 1e-6
_T = _B * _S
_NE = _TOPK * _T
_BS = 256
_NSLOT = _E * _CAP
_NW = 32


def _rope(x, cosf, sinf_signed):
    lane = lax.broadcasted_iota(jnp.int32, x.shape, 1)
    left = jnp.roll(x, -32, axis=1)
    right = jnp.roll(x, 32, axis=1)
    swap = jnp.where((lane % _HD) < (_HD // 2), left, right)
    return x * cosf + swap * sinf_signed


def _preattn_body(x_ref, wq_ref, wk_ref, wv_ref, bq_ref, bk_ref, bv_ref,
                  nw_ref, cq_ref, sq_ref,
                  q3_ref, k3_ref, v3_ref, k_ref, v_ref):
    x = x_ref[...]
    var = jnp.mean(x * x, axis=1, keepdims=True)
    h = (x * lax.rsqrt(var + _EPS) * nw_ref[...]).astype(jnp.bfloat16)
    q0 = jnp.dot(h, wq_ref[...].astype(jnp.bfloat16),
                 preferred_element_type=jnp.float32) + bq_ref[...]
    k0 = jnp.dot(h, wk_ref[...].astype(jnp.bfloat16),
                 preferred_element_type=jnp.float32) + bk_ref[...]
    v0 = jnp.dot(h, wv_ref[...].astype(jnp.bfloat16),
                 preferred_element_type=jnp.float32) + bv_ref[...]
    cos4 = jnp.concatenate([cq_ref[...]] * _HKV, axis=1)
    sin4 = jnp.concatenate([sq_ref[...]] * _HKV, axis=1)
    cos12 = jnp.concatenate([cos4] * (_HQ // _HKV), axis=1)
    sin12 = jnp.concatenate([sin4] * (_HQ // _HKV), axis=1)
    qro = _rope(q0, cos12, sin12)
    kro = _rope(k0, cos4, sin4)
    for hh in range(_HQ):
        q3_ref[hh] = qro[:, hh * _HD:(hh + 1) * _HD]
    for hh in range(_HKV):
        k3_ref[hh] = kro[:, hh * _HD:(hh + 1) * _HD]
        v3_ref[hh] = v0[:, hh * _HD:(hh + 1) * _HD]
    k_ref[...] = kro
    v_ref[...] = v0


def _preattn(x2d, p, cosq, sinq):
    full = lambda shape: pl.BlockSpec(shape, lambda i: (0,) * len(shape))
    row = lambda w: pl.BlockSpec((_BS, w), lambda i: (i, 0))
    return pl.pallas_call(
        _preattn_body,
        grid=(_S // _BS,),
        in_specs=[
            row(_D),
            full((_D, _HQ * _HD)), full((_D, _HKV * _HD)), full((_D, _HKV * _HD)),
            full((1, _HQ * _HD)), full((1, _HKV * _HD)), full((1, _HKV * _HD)),
            full((1, _D)),
            row(_HD), row(_HD),
        ],
        out_specs=[
            pl.BlockSpec((_HQ, _BS, _HD), lambda i: (0, i, 0)),
            pl.BlockSpec((_HKV, _BS, _HD), lambda i: (0, i, 0)),
            pl.BlockSpec((_HKV, _BS, _HD), lambda i: (0, i, 0)),
            row(_HKV * _HD), row(_HKV * _HD),
        ],
        out_shape=[
            jax.ShapeDtypeStruct((_HQ, _S, _HD), jnp.float32),
            jax.ShapeDtypeStruct((_HKV, _S, _HD), jnp.float32),
            jax.ShapeDtypeStruct((_HKV, _S, _HD), jnp.float32),
            jax.ShapeDtypeStruct((_S, _HKV * _HD), jnp.float32),
            jax.ShapeDtypeStruct((_S, _HKV * _HD), jnp.float32),
        ],
    )(x2d, p['Wq'], p['Wk'], p['Wv'],
      p['bq'].reshape(1, -1), p['bk'].reshape(1, -1), p['bv'].reshape(1, -1),
      p['attn_norm_w'].reshape(1, -1), cosq, sinq)


_AQ = 512


def _attn_body(q_ref, k_ref, v_ref, o_ref):
    i = pl.program_id(1)
    q = q_ref[0].astype(jnp.bfloat16)

    def branch(width):
        kb = k_ref[0, 0:width, :].astype(jnp.bfloat16)
        vb = v_ref[0, 0:width, :].astype(jnp.bfloat16)
        s = lax.dot_general(q, kb, (((1,), (1,)), ((), ())),
                            preferred_element_type=jnp.float32) * (1.0 / 8.0)
        rowi = i * _AQ + lax.broadcasted_iota(jnp.int32, s.shape, 0)
        coli = lax.broadcasted_iota(jnp.int32, s.shape, 1)
        s = jnp.where(coli <= rowi, s, -1e9)
        m = jnp.max(s, axis=1, keepdims=True)
        e = jnp.exp(s - m)
        o = lax.dot_general(e.astype(jnp.bfloat16), vb, (((1,), (0,)), ((), ())),
                            preferred_element_type=jnp.float32)
        o_ref[0] = o / jnp.sum(e, axis=1, keepdims=True)

    for bi in range(_S // _AQ):
        @pl.when(i == bi)
        def _(width=(bi + 1) * _AQ):
            branch(width)


def _attention(qT, kT, vT):
    rep = _HQ // _HKV
    return pl.pallas_call(
        _attn_body,
        grid=(_HQ, _S // _AQ),
        in_specs=[
            pl.BlockSpec((1, _AQ, _HD), lambda h, i: (h, i, 0)),
            pl.BlockSpec((1, _S, _HD), lambda h, i: (h // rep, 0, 0)),
            pl.BlockSpec((1, _S, _HD), lambda h, i: (h // rep, 0, 0)),
        ],
        out_specs=pl.BlockSpec((1, _AQ, _HD), lambda h, i: (h, i, 0)),
        out_shape=jax.ShapeDtypeStruct((_HQ, _S, _HD), jnp.float32),
    )(qT, kT, vT)


def _postattn_body(ao_ref, wo_ref, x_ref, nw_ref, wr_ref, x2_ref, h2_ref, lg_ref):
    aoc = jnp.concatenate([ao_ref[hh] for hh in range(_HQ)], axis=1)
    x2 = x_ref[...] + jnp.dot(aoc.astype(jnp.bfloat16),
                              wo_ref[...].astype(jnp.bfloat16),
                              preferred_element_type=jnp.float32)
    var = jnp.mean(x2 * x2, axis=1, keepdims=True)
    h2 = x2 * lax.rsqrt(var + _EPS) * nw_ref[...]
    x2_ref[...] = x2
    h2_ref[...] = h2
    lg_ref[...] = jnp.dot(h2, wr_ref[...], preferred_element_type=jnp.float32)


def _postattn(ao2, x2d, p):
    full = lambda shape: pl.BlockSpec(shape, lambda i: (0,) * len(shape))
    row = lambda w: pl.BlockSpec((_BS, w), lambda i: (i, 0))
    return pl.pallas_call(
        _postattn_body,
        grid=(_S // _BS,),
        in_specs=[pl.BlockSpec((_HQ, _BS, _HD), lambda i: (0, i, 0)),
                  full((_HQ * _HD, _D)), row(_D), full((1, _D)),
                  full((_D, _E))],
        out_specs=[row(_D), row(_D), row(_E)],
        out_shape=[
            jax.ShapeDtypeStruct((_S, _D), jnp.float32),
            jax.ShapeDtypeStruct((_S, _D), jnp.float32),
            jax.ShapeDtypeStruct((_S, _E), jnp.float32),
        ],
    )(ao2, p['Wo'], x2d, p['ffn_norm_w'].reshape(1, -1), p['Wr'])


def _route_body(lg_ref, dest_ref, wk_ref, aux_ref, counts_ref, psum_ref):
    b = pl.program_id(0)

    @pl.when(b == 0)
    def _():
        counts_ref[...] = jnp.zeros((1, _E), jnp.float32)
        psum_ref[...] = jnp.zeros((1, _E), jnp.float32)

    lg = lg_ref[...]
    m = jnp.max(lg, axis=1, keepdims=True)
    ex = jnp.exp(lg - m)
    prob = ex / jnp.sum(ex, axis=1, keepdims=True)

    @pl.when(b < _S // _BS)
    def _():
        psum_ref[...] += jnp.sum(prob, axis=0, keepdims=True)

    ie = lax.broadcasted_iota(jnp.int32, (_BS, _E), 1)
    m1 = jnp.max(prob, axis=1, keepdims=True)
    i1 = jnp.min(jnp.where(prob >= m1, ie, _E), axis=1, keepdims=True)
    p2 = jnp.where(ie == i1, -1.0, prob)
    m2 = jnp.max(p2, axis=1, keepdims=True)
    i2 = jnp.min(jnp.where(p2 >= m2, ie, _E), axis=1, keepdims=True)
    den = m1 + m2
    c = b // (_S // _BS)
    fe = jnp.where(c == 0, i1, i2)
    w = jnp.where(c == 0, m1, m2) / den
    oh = (ie == fe).astype(jnp.float32)
    ri = lax.broadcasted_iota(jnp.int32, (_BS, _BS), 0)
    ci = lax.broadcasted_iota(jnp.int32, (_BS, _BS), 1)
    ltri = (ci < ri).astype(jnp.bfloat16)
    before = jnp.dot(ltri, oh.astype(jnp.bfloat16),
                     preferred_element_type=jnp.float32)
    pos = jnp.sum((counts_ref[...] + before) * oh, axis=1,
                  keepdims=True).astype(jnp.int32)
    counts_ref[...] += jnp.sum(oh, axis=0, keepdims=True)
    keep = pos < _CAP
    posc = jnp.minimum(pos, _CAP - 1)
    entry = b * _BS + lax.broadcasted_iota(jnp.int32, (_BS, 1), 0)
    dest_ref[...] = jnp.where(keep, fe * _CAP + posc, _NSLOT + (entry & (_CAP - 1)))
    wk_ref[...] = jnp.where(keep, w, 0.0)

    @pl.when(b == _NE // _BS - 1)
    def _():
        aux_ref[...] = (_E * jnp.sum(counts_ref[...] * psum_ref[...],
                                     axis=1, keepdims=True)
                        / (float(_NE) * float(_T)))


def _route(logits):
    nb = _NE // _BS
    return pl.pallas_call(
        _route_body,
        grid=(nb,),
        in_specs=[pl.BlockSpec((_BS, _E), lambda b: (b % (_S // _BS), 0))],
        out_specs=[
            pl.BlockSpec((_BS, 1), lambda b: (b, 0)),
            pl.BlockSpec((_BS, 1), lambda b: (b, 0)),
            pl.BlockSpec((1, 1), lambda b: (0, 0)),
        ],
        out_shape=[
            jax.ShapeDtypeStruct((_NE, 1), jnp.int32),
            jax.ShapeDtypeStruct((_NE, 1), jnp.float32),
            jax.ShapeDtypeStruct((1, 1), jnp.float32),
        ],
        scratch_shapes=[pltpu.VMEM((1, _E), jnp.float32),
                        pltpu.VMEM((1, _E), jnp.float32)],
    )(logits)


def _sc_mesh():
    return plsc.VectorSubcoreMesh(core_axis_name="c", subcore_axis_name="s")


def _build_tables(dest, wk):
    per_w = _NSLOT // _NW

    @functools.partial(
        pl.kernel,
        out_type=(jax.ShapeDtypeStruct((_NSLOT,), jnp.int32),
                  jax.ShapeDtypeStruct((_NSLOT,), jnp.float32)),
        mesh=_sc_mesh(),
        scratch_types=[pltpu.VMEM((_NE,), jnp.int32),
                       pltpu.VMEM((_NE,), jnp.float32),
                       pltpu.VMEM((per_w,), jnp.int32),
                       pltpu.VMEM((per_w,), jnp.float32)],
        compiler_params=pltpu.CompilerParams(needs_layout_passes=False),
    )
    def k(dest_hbm, wk_hbm, src_hbm, sw_hbm, dest_v, wk_v, src_l, sw_l):
        wid = lax.axis_index("s") * 2 + lax.axis_index("c")
        lo = wid * per_w
        pltpu.sync_copy(dest_hbm, dest_v)
        pltpu.sync_copy(wk_hbm, wk_v)
        for i in range(per_w // 16):
            evec = lo + i * 16 + jnp.arange(16, dtype=jnp.int32)
            src_l[pl.ds(i * 16, 16)] = evec & (_T - 1)
            sw_l[pl.ds(i * 16, 16)] = jnp.zeros((16,), jnp.float32)

        def body(i, carry):
            d = dest_v[pl.ds(i * 16, 16)]
            w = wk_v[pl.ds(i * 16, 16)]
            evec = i * 16 + jnp.arange(16, dtype=jnp.int32)
            tok = jnp.where(evec >= _T, evec - _T, evec)
            msk = (d >= lo) & (d < lo + per_w)
            plsc.store_scatter(src_l, [d - lo], tok, mask=msk)
            plsc.store_scatter(sw_l, [d - lo], w, mask=msk)
            return carry

        lax.fori_loop(0, _NE // 16, body, 0)
        pltpu.sync_copy(src_l, src_hbm.at[pl.ds(lo, per_w)])
        pltpu.sync_copy(sw_l, sw_hbm.at[pl.ds(lo, per_w)])

    return k(dest, wk)


def _sc_gather(table, idx, n_rows, chunk):
    per_w = n_rows // _NW
    nch = per_w // chunk

    @functools.partial(
        pl.kernel,
        out_type=jax.ShapeDtypeStruct((n_rows, _D), jnp.float32),
        mesh=_sc_mesh(),
        scratch_types=[pltpu.VMEM((chunk,), jnp.int32),
                       pltpu.VMEM((chunk,), jnp.int32),
                       pltpu.VMEM((chunk, _D), jnp.float32),
                       pltpu.VMEM((chunk, _D), jnp.float32),
                       pltpu.SemaphoreType.DMA,
                       pltpu.SemaphoreType.DMA],
        compiler_params=pltpu.CompilerParams(needs_layout_passes=False),
    )
    def k(table_hbm, idx_hbm, out_hbm, ia, ib, ra, rb, sa, sb):
        wid = lax.axis_index("s") * 2 + lax.axis_index("c")
        base = wid * per_w
        bufs = [(ia, ra, sa), (ib, rb, sb)]
        copies = [None] * nch
        for ci in range(nch):
            iv, rv, sem = bufs[ci % 2]
            pltpu.sync_copy(idx_hbm.at[pl.ds(base + ci * chunk, chunk)], iv)
            copies[ci] = pltpu.async_copy(table_hbm.at[iv], rv, sem)
            if ci >= 1:
                pv = bufs[(ci - 1) % 2][1]
                copies[ci - 1].wait()
                pltpu.sync_copy(
                    pv, out_hbm.at[pl.ds(base + (ci - 1) * chunk, chunk)])
        copies[nch - 1].wait()
        pltpu.sync_copy(
            bufs[(nch - 1) % 2][1],
            out_hbm.at[pl.ds(base + (nch - 1) * chunk, chunk)])

    return k(table, idx)


def _expert_body(ein_ref, wg_ref, wu_ref, wd_ref, sw_ref, out_ref):
    e = pl.program_id(0)

    @pl.when(e < _E)
    def _():
        xin = ein_ref[0].astype(jnp.bfloat16)
        g = jnp.dot(xin, wg_ref[0].astype(jnp.bfloat16),
                    preferred_element_type=jnp.float32)
        u = jnp.dot(xin, wu_ref[0].astype(jnp.bfloat16),
                    preferred_element_type=jnp.float32)
        a = (g * lax.logistic(g) * u).astype(jnp.bfloat16)
        o = jnp.dot(a, wd_ref[0].astype(jnp.bfloat16),
                    preferred_element_type=jnp.float32)
        out_ref[0] = o * sw_ref[0]

    @pl.when(e == _E)
    def _():
        out_ref[0] = jnp.zeros((_CAP, _D), jnp.float32)


def _experts(ein3, sw3, p):
    cl = lambda e: (jnp.minimum(e, _E - 1), 0, 0)
    return pl.pallas_call(
        _expert_body,
        grid=(_E + 1,),
        in_specs=[
            pl.BlockSpec((1, _CAP, _D), cl),
            pl.BlockSpec((1, _D, _FF), cl),
            pl.BlockSpec((1, _D, _FF), cl),
            pl.BlockSpec((1, _FF, _D), cl),
            pl.BlockSpec((1, _CAP, 1), cl),
        ],
        out_specs=pl.BlockSpec((1, _CAP, _D), lambda e: (e, 0, 0)),
        out_shape=jax.ShapeDtypeStruct((_E + 1, _CAP, _D), jnp.float32),
    )(ein3, p['Wg'], p['Wu'], p['Wd'], sw3)


def _shared_body(h2_ref, sg_ref, su_ref, sd_ref, o_ref):
    h2 = h2_ref[...].astype(jnp.bfloat16)
    g = jnp.dot(h2, sg_ref[...].astype(jnp.bfloat16),
                preferred_element_type=jnp.float32)
    u = jnp.dot(h2, su_ref[...].astype(jnp.bfloat16),
                preferred_element_type=jnp.float32)
    o_ref[...] = jnp.dot((g * lax.logistic(g) * u).astype(jnp.bfloat16),
                         sd_ref[...].astype(jnp.bfloat16),
                         preferred_element_type=jnp.float32)


def _shared(h2, p):
    full = lambda shape: pl.BlockSpec(shape, lambda i: (0,) * len(shape))
    row = lambda w: pl.BlockSpec((_BS, w), lambda i: (i, 0))
    return pl.pallas_call(
        _shared_body,
        grid=(_S // _BS,),
        in_specs=[row(_D), full((_D, _SFF)), full((_D, _SFF)),
                  full((_SFF, _D))],
        out_specs=row(_D),
        out_shape=jax.ShapeDtypeStruct((_S, _D), jnp.float32),
    )(h2, p['Sg'], p['Su'], p['Sd'])


def _final_body(x2_ref, ta_ref, tb_ref, sh_ref, o_ref):
    o_ref[...] = x2_ref[...] + ta_ref[...] + tb_ref[...] + sh_ref[...]


def _final(x2, tok, shared):
    row = lambda w: pl.BlockSpec((_BS, w), lambda i: (i, 0))
    nb = _S // _BS
    return pl.pallas_call(
        _final_body,
        grid=(nb,),
        in_specs=[
            row(_D),
            pl.BlockSpec((_BS, _D), lambda i: (i, 0)),
            pl.BlockSpec((_BS, _D), lambda i: (i + nb, 0)),
            row(_D),
        ],
        out_specs=row(_D),
        out_shape=jax.ShapeDtypeStruct((_S, _D), jnp.float32),
    )(x2, tok, tok, shared)


def kernel(x, rope_cos, rope_sin, params):
    p = params
    x2d = x.reshape(_S, _D)
    sign = jnp.concatenate([-jnp.ones((_HD // 2,), jnp.float32),
                            jnp.ones((_HD // 2,), jnp.float32)])
    cosq = rope_cos
    sinq = rope_sin * sign[None, :]

    qT, kT, vT, kk, v = _preattn(x2d, p, cosq, sinq)
    kv_k = kk.reshape(_B, _S, _HKV, _HD)
    kv_v = v.reshape(_B, _S, _HKV, _HD)

    ao = _attention(qT, kT, vT)

    x2, h2, logits = _postattn(ao, x2d, p)
    shared = _shared(h2, p)
    dest2, wk2, aux = _route(logits)
    dest = dest2.reshape(_NE)
    wk = wk2.reshape(_NE)

    src, sw = _build_tables(dest, wk)

    ein = _sc_gather(h2, src, _NSLOT, 64)
    eout = _experts(ein.reshape(_E, _CAP, _D), sw.reshape(_E, _CAP, 1), p)
    eoutp = eout.reshape((_E + 1) * _CAP, _D)
    tok = _sc_gather(eoutp, dest, _NE, 64)

    out = _final(x2, tok, shared)
    return out.reshape(_B, _S, _D), (kv_k, kv_v), aux.reshape(())


# speedup vs baseline: 2.0504x; 1.0241x over previous
import functools

import jax
import jax.numpy as jnp
from jax import lax
from jax.experimental import pallas as pl
from jax.experimental.pallas import tpu as pltpu
from jax.experimental.pallas import tpu_sc as plsc

_B, _S, _D = 1, 2048, 768
_HQ, _HKV, _HD = 12, 4, 64
_E, _FF, _TOPK, _CAP = 64, 512, 2, 128
_SFF = 2048
_EPS = 1e-6
_T = _B * _S
_NE = _TOPK * _T
_BS = 512
_NSLOT = _E * _CAP
_NW = 32


def _rope(x, cosf, sinf_signed):
    lane = lax.broadcasted_iota(jnp.int32, x.shape, 1)
    left = jnp.roll(x, -32, axis=1)
    right = jnp.roll(x, 32, axis=1)
    swap = jnp.where((lane % _HD) < (_HD // 2), left, right)
    return x * cosf + swap * sinf_signed


def _preattn_body(x_ref, wq_ref, wk_ref, wv_ref, bq_ref, bk_ref, bv_ref,
                  nw_ref, cq_ref, sq_ref,
                  q3_ref, k3_ref, v3_ref, k_ref, v_ref):
    x = x_ref[...]
    var = jnp.mean(x * x, axis=1, keepdims=True)
    h = (x * lax.rsqrt(var + _EPS) * nw_ref[...]).astype(jnp.bfloat16)
    q0 = jnp.dot(h, wq_ref[...].astype(jnp.bfloat16),
                 preferred_element_type=jnp.float32) + bq_ref[...]
    k0 = jnp.dot(h, wk_ref[...].astype(jnp.bfloat16),
                 preferred_element_type=jnp.float32) + bk_ref[...]
    v0 = jnp.dot(h, wv_ref[...].astype(jnp.bfloat16),
                 preferred_element_type=jnp.float32) + bv_ref[...]
    cos4 = jnp.concatenate([cq_ref[...]] * _HKV, axis=1)
    sin4 = jnp.concatenate([sq_ref[...]] * _HKV, axis=1)
    cos12 = jnp.concatenate([cos4] * (_HQ // _HKV), axis=1)
    sin12 = jnp.concatenate([sin4] * (_HQ // _HKV), axis=1)
    qro = _rope(q0, cos12, sin12)
    kro = _rope(k0, cos4, sin4)
    for hh in range(_HQ):
        q3_ref[hh] = qro[:, hh * _HD:(hh + 1) * _HD]
    for hh in range(_HKV):
        k3_ref[hh] = kro[:, hh * _HD:(hh + 1) * _HD]
        v3_ref[hh] = v0[:, hh * _HD:(hh + 1) * _HD]
    k_ref[...] = kro
    v_ref[...] = v0


def _preattn(x2d, p, cosq, sinq):
    full = lambda shape: pl.BlockSpec(shape, lambda i: (0,) * len(shape))
    row = lambda w: pl.BlockSpec((_BS, w), lambda i: (i, 0))
    return pl.pallas_call(
        _preattn_body,
        grid=(_S // _BS,),
        in_specs=[
            row(_D),
            full((_D, _HQ * _HD)), full((_D, _HKV * _HD)), full((_D, _HKV * _HD)),
            full((1, _HQ * _HD)), full((1, _HKV * _HD)), full((1, _HKV * _HD)),
            full((1, _D)),
            row(_HD), row(_HD),
        ],
        out_specs=[
            pl.BlockSpec((_HQ, _BS, _HD), lambda i: (0, i, 0)),
            pl.BlockSpec((_HKV, _BS, _HD), lambda i: (0, i, 0)),
            pl.BlockSpec((_HKV, _BS, _HD), lambda i: (0, i, 0)),
            row(_HKV * _HD), row(_HKV * _HD),
        ],
        out_shape=[
            jax.ShapeDtypeStruct((_HQ, _S, _HD), jnp.float32),
            jax.ShapeDtypeStruct((_HKV, _S, _HD), jnp.float32),
            jax.ShapeDtypeStruct((_HKV, _S, _HD), jnp.float32),
            jax.ShapeDtypeStruct((_S, _HKV * _HD), jnp.float32),
            jax.ShapeDtypeStruct((_S, _HKV * _HD), jnp.float32),
        ],
    )(x2d, p['Wq'], p['Wk'], p['Wv'],
      p['bq'].reshape(1, -1), p['bk'].reshape(1, -1), p['bv'].reshape(1, -1),
      p['attn_norm_w'].reshape(1, -1), cosq, sinq)


_AQ = 512


def _attn_body(q_ref, k_ref, v_ref, o_ref):
    i = pl.program_id(1)
    q = q_ref[0].astype(jnp.bfloat16)

    def branch(width):
        kb = k_ref[0, 0:width, :].astype(jnp.bfloat16)
        vb = v_ref[0, 0:width, :].astype(jnp.bfloat16)
        s = lax.dot_general(q, kb, (((1,), (1,)), ((), ())),
                            preferred_element_type=jnp.float32) * (1.0 / 8.0)
        rowi = i * _AQ + lax.broadcasted_iota(jnp.int32, s.shape, 0)
        coli = lax.broadcasted_iota(jnp.int32, s.shape, 1)
        s = jnp.where(coli <= rowi, s, -1e9)
        m = jnp.max(s, axis=1, keepdims=True)
        e = jnp.exp(s - m)
        o = lax.dot_general(e.astype(jnp.bfloat16), vb, (((1,), (0,)), ((), ())),
                            preferred_element_type=jnp.float32)
        o_ref[0] = o / jnp.sum(e, axis=1, keepdims=True)

    for bi in range(_S // _AQ):
        @pl.when(i == bi)
        def _(width=(bi + 1) * _AQ):
            branch(width)


def _attention(qT, kT, vT):
    rep = _HQ // _HKV
    return pl.pallas_call(
        _attn_body,
        grid=(_HQ, _S // _AQ),
        in_specs=[
            pl.BlockSpec((1, _AQ, _HD), lambda h, i: (h, i, 0)),
            pl.BlockSpec((1, _S, _HD), lambda h, i: (h // rep, 0, 0)),
            pl.BlockSpec((1, _S, _HD), lambda h, i: (h // rep, 0, 0)),
        ],
        out_specs=pl.BlockSpec((1, _AQ, _HD), lambda h, i: (h, i, 0)),
        out_shape=jax.ShapeDtypeStruct((_HQ, _S, _HD), jnp.float32),
    )(qT, kT, vT)


def _postattn_body(ao_ref, wo_ref, x_ref, nw_ref, wr_ref, x2_ref, h2_ref, lg_ref):
    aoc = jnp.concatenate([ao_ref[hh] for hh in range(_HQ)], axis=1)
    x2 = x_ref[...] + jnp.dot(aoc.astype(jnp.bfloat16),
                              wo_ref[...].astype(jnp.bfloat16),
                              preferred_element_type=jnp.float32)
    var = jnp.mean(x2 * x2, axis=1, keepdims=True)
    h2 = x2 * lax.rsqrt(var + _EPS) * nw_ref[...]
    x2_ref[...] = x2
    h2_ref[...] = h2
    lg_ref[...] = jnp.dot(h2, wr_ref[...], preferred_element_type=jnp.float32)


def _postattn(ao2, x2d, p):
    full = lambda shape: pl.BlockSpec(shape, lambda i: (0,) * len(shape))
    row = lambda w: pl.BlockSpec((_BS, w), lambda i: (i, 0))
    return pl.pallas_call(
        _postattn_body,
        grid=(_S // _BS,),
        in_specs=[pl.BlockSpec((_HQ, _BS, _HD), lambda i: (0, i, 0)),
                  full((_HQ * _HD, _D)), row(_D), full((1, _D)),
                  full((_D, _E))],
        out_specs=[row(_D), row(_D), row(_E)],
        out_shape=[
            jax.ShapeDtypeStruct((_S, _D), jnp.float32),
            jax.ShapeDtypeStruct((_S, _D), jnp.float32),
            jax.ShapeDtypeStruct((_S, _E), jnp.float32),
        ],
    )(ao2, p['Wo'], x2d, p['ffn_norm_w'].reshape(1, -1), p['Wr'])


def _route_body(lg_ref, dest_ref, wk_ref, aux_ref, counts_ref, psum_ref):
    b = pl.program_id(0)

    @pl.when(b == 0)
    def _():
        counts_ref[...] = jnp.zeros((1, _E), jnp.float32)
        psum_ref[...] = jnp.zeros((1, _E), jnp.float32)

    lg = lg_ref[...]
    m = jnp.max(lg, axis=1, keepdims=True)
    ex = jnp.exp(lg - m)
    prob = ex / jnp.sum(ex, axis=1, keepdims=True)

    @pl.when(b < _S // _BS)
    def _():
        psum_ref[...] += jnp.sum(prob, axis=0, keepdims=True)

    ie = lax.broadcasted_iota(jnp.int32, (_BS, _E), 1)
    m1 = jnp.max(prob, axis=1, keepdims=True)
    i1 = jnp.min(jnp.where(prob >= m1, ie, _E), axis=1, keepdims=True)
    p2 = jnp.where(ie == i1, -1.0, prob)
    m2 = jnp.max(p2, axis=1, keepdims=True)
    i2 = jnp.min(jnp.where(p2 >= m2, ie, _E), axis=1, keepdims=True)
    den = m1 + m2
    c = b // (_S // _BS)
    fe = jnp.where(c == 0, i1, i2)
    w = jnp.where(c == 0, m1, m2) / den
    oh = (ie == fe).astype(jnp.float32)
    ri = lax.broadcasted_iota(jnp.int32, (_BS, _BS), 0)
    ci = lax.broadcasted_iota(jnp.int32, (_BS, _BS), 1)
    ltri = (ci < ri).astype(jnp.bfloat16)
    before = jnp.dot(ltri, oh.astype(jnp.bfloat16),
                     preferred_element_type=jnp.float32)
    pos = jnp.sum((counts_ref[...] + before) * oh, axis=1,
                  keepdims=True).astype(jnp.int32)
    counts_ref[...] += jnp.sum(oh, axis=0, keepdims=True)
    keep = pos < _CAP
    posc = jnp.minimum(pos, _CAP - 1)
    entry = b * _BS + lax.broadcasted_iota(jnp.int32, (_BS, 1), 0)
    dest_ref[...] = jnp.where(keep, fe * _CAP + posc, _NSLOT + (entry & (_CAP - 1)))
    wk_ref[...] = jnp.where(keep, w, 0.0)

    @pl.when(b == _NE // _BS - 1)
    def _():
        aux_ref[...] = (_E * jnp.sum(counts_ref[...] * psum_ref[...],
                                     axis=1, keepdims=True)
                        / (float(_NE) * float(_T)))


def _route(logits):
    nb = _NE // _BS
    return pl.pallas_call(
        _route_body,
        grid=(nb,),
        in_specs=[pl.BlockSpec((_BS, _E), lambda b: (b % (_S // _BS), 0))],
        out_specs=[
            pl.BlockSpec((_BS, 1), lambda b: (b, 0)),
            pl.BlockSpec((_BS, 1), lambda b: (b, 0)),
            pl.BlockSpec((1, 1), lambda b: (0, 0)),
        ],
        out_shape=[
            jax.ShapeDtypeStruct((_NE, 1), jnp.int32),
            jax.ShapeDtypeStruct((_NE, 1), jnp.float32),
            jax.ShapeDtypeStruct((1, 1), jnp.float32),
        ],
        scratch_shapes=[pltpu.VMEM((1, _E), jnp.float32),
                        pltpu.VMEM((1, _E), jnp.float32)],
    )(logits)


def _sc_mesh():
    return plsc.VectorSubcoreMesh(core_axis_name="c", subcore_axis_name="s")


def _build_tables(dest, wk):
    per_w = _NSLOT // _NW

    @functools.partial(
        pl.kernel,
        out_type=(jax.ShapeDtypeStruct((_NSLOT,), jnp.int32),
                  jax.ShapeDtypeStruct((_NSLOT,), jnp.float32)),
        mesh=_sc_mesh(),
        scratch_types=[pltpu.VMEM((_NE,), jnp.int32),
                       pltpu.VMEM((_NE,), jnp.float32),
                       pltpu.VMEM((per_w,), jnp.int32),
                       pltpu.VMEM((per_w,), jnp.float32)],
        compiler_params=pltpu.CompilerParams(needs_layout_passes=False),
    )
    def k(dest_hbm, wk_hbm, src_hbm, sw_hbm, dest_v, wk_v, src_l, sw_l):
        wid = lax.axis_index("s") * 2 + lax.axis_index("c")
        lo = wid * per_w
        pltpu.sync_copy(dest_hbm, dest_v)
        pltpu.sync_copy(wk_hbm, wk_v)
        for i in range(per_w // 16):
            evec = lo + i * 16 + jnp.arange(16, dtype=jnp.int32)
            src_l[pl.ds(i * 16, 16)] = evec & (_T - 1)
            sw_l[pl.ds(i * 16, 16)] = jnp.zeros((16,), jnp.float32)

        def body(i, carry):
            d = dest_v[pl.ds(i * 16, 16)]
            w = wk_v[pl.ds(i * 16, 16)]
            evec = i * 16 + jnp.arange(16, dtype=jnp.int32)
            tok = jnp.where(evec >= _T, evec - _T, evec)
            msk = (d >= lo) & (d < lo + per_w)
            plsc.store_scatter(src_l, [d - lo], tok, mask=msk)
            plsc.store_scatter(sw_l, [d - lo], w, mask=msk)
            return carry

        lax.fori_loop(0, _NE // 16, body, 0)
        pltpu.sync_copy(src_l, src_hbm.at[pl.ds(lo, per_w)])
        pltpu.sync_copy(sw_l, sw_hbm.at[pl.ds(lo, per_w)])

    return k(dest, wk)


def _sc_gather(table, idx, n_rows, chunk):
    per_w = n_rows // _NW
    nch = per_w // chunk

    @functools.partial(
        pl.kernel,
        out_type=jax.ShapeDtypeStruct((n_rows, _D), jnp.float32),
        mesh=_sc_mesh(),
        scratch_types=[pltpu.VMEM((chunk,), jnp.int32),
                       pltpu.VMEM((chunk,), jnp.int32),
                       pltpu.VMEM((chunk, _D), jnp.float32),
                       pltpu.VMEM((chunk, _D), jnp.float32),
                       pltpu.SemaphoreType.DMA,
                       pltpu.SemaphoreType.DMA],
        compiler_params=pltpu.CompilerParams(needs_layout_passes=False),
    )
    def k(table_hbm, idx_hbm, out_hbm, ia, ib, ra, rb, sa, sb):
        wid = lax.axis_index("s") * 2 + lax.axis_index("c")
        base = wid * per_w
        bufs = [(ia, ra, sa), (ib, rb, sb)]
        copies = [None] * nch
        for ci in range(nch):
            iv, rv, sem = bufs[ci % 2]
            pltpu.sync_copy(idx_hbm.at[pl.ds(base + ci * chunk, chunk)], iv)
            copies[ci] = pltpu.async_copy(table_hbm.at[iv], rv, sem)
            if ci >= 1:
                pv = bufs[(ci - 1) % 2][1]
                copies[ci - 1].wait()
                pltpu.sync_copy(
                    pv, out_hbm.at[pl.ds(base + (ci - 1) * chunk, chunk)])
        copies[nch - 1].wait()
        pltpu.sync_copy(
            bufs[(nch - 1) % 2][1],
            out_hbm.at[pl.ds(base + (nch - 1) * chunk, chunk)])

    return k(table, idx)


def _expert_body(ein_ref, wg_ref, wu_ref, wd_ref, sw_ref, out_ref):
    e = pl.program_id(0)

    @pl.when(e < _E)
    def _():
        xin = ein_ref[0].astype(jnp.bfloat16)
        g = jnp.dot(xin, wg_ref[0].astype(jnp.bfloat16),
                    preferred_element_type=jnp.float32)
        u = jnp.dot(xin, wu_ref[0].astype(jnp.bfloat16),
                    preferred_element_type=jnp.float32)
        a = (g * lax.logistic(g) * u).astype(jnp.bfloat16)
        o = jnp.dot(a, wd_ref[0].astype(jnp.bfloat16),
                    preferred_element_type=jnp.float32)
        out_ref[0] = o * sw_ref[0]

    @pl.when(e == _E)
    def _():
        out_ref[0] = jnp.zeros((_CAP, _D), jnp.float32)


def _experts(ein3, sw3, p):
    cl = lambda e: (jnp.minimum(e, _E - 1), 0, 0)
    return pl.pallas_call(
        _expert_body,
        grid=(_E + 1,),
        in_specs=[
            pl.BlockSpec((1, _CAP, _D), cl),
            pl.BlockSpec((1, _D, _FF), cl),
            pl.BlockSpec((1, _D, _FF), cl),
            pl.BlockSpec((1, _FF, _D), cl),
            pl.BlockSpec((1, _CAP, 1), cl),
        ],
        out_specs=pl.BlockSpec((1, _CAP, _D), lambda e: (e, 0, 0)),
        out_shape=jax.ShapeDtypeStruct((_E + 1, _CAP, _D), jnp.float32),
    )(ein3, p['Wg'], p['Wu'], p['Wd'], sw3)


def _shared_body(h2_ref, sg_ref, su_ref, sd_ref, o_ref):
    h2 = h2_ref[...].astype(jnp.bfloat16)
    g = jnp.dot(h2, sg_ref[...].astype(jnp.bfloat16),
                preferred_element_type=jnp.float32)
    u = jnp.dot(h2, su_ref[...].astype(jnp.bfloat16),
                preferred_element_type=jnp.float32)
    o_ref[...] = jnp.dot((g * lax.logistic(g) * u).astype(jnp.bfloat16),
                         sd_ref[...].astype(jnp.bfloat16),
                         preferred_element_type=jnp.float32)


def _shared(h2, p):
    full = lambda shape: pl.BlockSpec(shape, lambda i: (0,) * len(shape))
    row = lambda w: pl.BlockSpec((_BS, w), lambda i: (i, 0))
    return pl.pallas_call(
        _shared_body,
        grid=(_S // _BS,),
        in_specs=[row(_D), full((_D, _SFF)), full((_D, _SFF)),
                  full((_SFF, _D))],
        out_specs=row(_D),
        out_shape=jax.ShapeDtypeStruct((_S, _D), jnp.float32),
    )(h2, p['Sg'], p['Su'], p['Sd'])


def _final_body(x2_ref, ta_ref, tb_ref, sh_ref, o_ref):
    o_ref[...] = x2_ref[...] + ta_ref[...] + tb_ref[...] + sh_ref[...]


def _final(x2, tok, shared):
    row = lambda w: pl.BlockSpec((_BS, w), lambda i: (i, 0))
    nb = _S // _BS
    return pl.pallas_call(
        _final_body,
        grid=(nb,),
        in_specs=[
            row(_D),
            pl.BlockSpec((_BS, _D), lambda i: (i, 0)),
            pl.BlockSpec((_BS, _D), lambda i: (i + nb, 0)),
            row(_D),
        ],
        out_specs=row(_D),
        out_shape=jax.ShapeDtypeStruct((_S, _D), jnp.float32),
    )(x2, tok, tok, shared)


def kernel(x, rope_cos, rope_sin, params):
    p = params
    x2d = x.reshape(_S, _D)
    sign = jnp.concatenate([-jnp.ones((_HD // 2,), jnp.float32),
                            jnp.ones((_HD // 2,), jnp.float32)])
    cosq = rope_cos
    sinq = rope_sin * sign[None, :]

    qT, kT, vT, kk, v = _preattn(x2d, p, cosq, sinq)
    kv_k = kk.reshape(_B, _S, _HKV, _HD)
    kv_v = v.reshape(_B, _S, _HKV, _HD)

    ao = _attention(qT, kT, vT)

    x2, h2, logits = _postattn(ao, x2d, p)
    shared = _shared(h2, p)
    dest2, wk2, aux = _route(logits)
    dest = dest2.reshape(_NE)
    wk = wk2.reshape(_NE)

    src, sw = _build_tables(dest, wk)

    ein = _sc_gather(h2, src, _NSLOT, 64)
    eout = _experts(ein.reshape(_E, _CAP, _D), sw.reshape(_E, _CAP, 1), p)
    eoutp = eout.reshape((_E + 1) * _CAP, _D)
    tok = _sc_gather(eoutp, dest, _NE, 64)

    out = _final(x2, tok, shared)
    return out.reshape(_B, _S, _D), (kv_k, kv_v), aux.reshape(())
